# baseline (speedup 1.0000x reference)
import functools

import jax
import jax.numpy as jnp
from jax import lax
from jax.experimental import pallas as pl
from jax.experimental.pallas import tpu as pltpu
from jax.experimental.pallas import tpu_sc as plsc

N = 10000
E = 320000
H = 128

NC = 2
NS = 16
NW = NC * NS
EPW = E // NW
K = 40
NCHUNK = EPW // K
NP = 10240
RPS = NP // NS

_mesh = plsc.VectorSubcoreMesh(core_axis_name="c", subcore_axis_name="s")


def _worker_base():
    c = lax.axis_index("c")
    s = lax.axis_index("s")
    return c, s, (s * NC + c) * EPW


def _zero_rows(buf, rows):
    zv = jnp.zeros((16,), jnp.float32)

    @pl.loop(0, rows)
    def _(r):
        for c in range(H // 16):
            buf[r, pl.ds(16 * c, 16)] = zv


def _acc_init(acc, zbuf, rows, s):
    _zero_rows(zbuf, rows)
    base = s * RPS
    for j in range(RPS // rows):
        pltpu.sync_copy(zbuf, acc.at[pl.ds(base + j * rows, rows)])
    rem = RPS % rows
    if rem:
        pltpu.sync_copy(zbuf.at[pl.ds(0, rem)],
                        acc.at[pl.ds(base + (RPS // rows) * rows, rem)])


def _acc_writeout(acc, msg_hbm, rows, c, s):
    base = s * RPS
    for j in range(RPS // rows):
        pltpu.sync_copy(acc.at[pl.ds(base + j * rows, rows)],
                        msg_hbm.at[c, pl.ds(base + j * rows, rows)])
    rem = RPS % rows
    if rem:
        pltpu.sync_copy(acc.at[pl.ds(base + (RPS // rows) * rows, rem)],
                        msg_hbm.at[c, pl.ds(base + (RPS // rows) * rows, rem)])


@functools.partial(
    pl.kernel,
    mesh=_mesh,
    out_type=[
        jax.ShapeDtypeStruct((E, H), jnp.float32),
        jax.ShapeDtypeStruct((E, H), jnp.float32),
    ],
    scratch_types=[
        [pltpu.VMEM((K,), jnp.int32)] * 2,
        [pltpu.VMEM((K,), jnp.int32)] * 2,
        [pltpu.VMEM((K, 2 * H), jnp.float32)] * 2,
        [pltpu.VMEM((K, 2 * H), jnp.float32)] * 2,
        [pltpu.VMEM((K, H), jnp.float32)] * 2,
        [pltpu.VMEM((K, H), jnp.float32)] * 2,
        pltpu.VMEM((H,), jnp.float32),
        [pltpu.SemaphoreType.DMA] * 2,
        [pltpu.SemaphoreType.DMA] * 2,
    ],
)
def _sc_g1(nsd_h, ndb_h, src_h, dst_h, bee_h,
           e1_h, z2_h,
           idx_s, idx_d, gin_s, gin_d, oe, oz, bias_v, gsem, osem):
    c, s, base = _worker_base()

    pltpu.sync_copy(bee_h, bias_v)

    def issue_in(sl, ch):
        b0 = base + ch * K
        pltpu.sync_copy(src_h.at[pl.ds(b0, K)], idx_s[sl])
        pltpu.sync_copy(dst_h.at[pl.ds(b0, K)], idx_d[sl])
        pltpu.async_copy(nsd_h.at[idx_s[sl]], gin_s[sl], gsem[sl])
        pltpu.async_copy(ndb_h.at[idx_d[sl]], gin_d[sl], gsem[sl])

    def wait_in(sl):
        pltpu.make_async_copy(nsd_h.at[idx_s[sl]], gin_s[sl], gsem[sl]).wait()
        pltpu.make_async_copy(ndb_h.at[idx_d[sl]], gin_d[sl], gsem[sl]).wait()

    def compute(sl):
        @pl.loop(0, K)
        def _(r):
            for cc in range(H // 16):
                o = 16 * cc
                vs = gin_s[sl][r, pl.ds(o, 16)]
                vd = gin_d[sl][r, pl.ds(o, 16)]
                bv = bias_v[pl.ds(o, 16)]
                oe[sl][r, pl.ds(o, 16)] = jnp.maximum(vs + vd + bv, 0.0)
                va = gin_s[sl][r, pl.ds(H + o, 16)]
                vb = gin_d[sl][r, pl.ds(H + o, 16)]
                oz[sl][r, pl.ds(o, 16)] = va + vb

    def issue_out(sl, ch):
        b0 = base + ch * K
        pltpu.async_copy(oe[sl], e1_h.at[pl.ds(b0, K)], osem[sl])
        pltpu.async_copy(oz[sl], z2_h.at[pl.ds(b0, K)], osem[sl])

    def wait_out(sl, ch):
        b0 = base + ch * K
        pltpu.make_async_copy(oe[sl], e1_h.at[pl.ds(b0, K)], osem[sl]).wait()
        pltpu.make_async_copy(oz[sl], z2_h.at[pl.ds(b0, K)], osem[sl]).wait()

    for sl in (0, 1):
        issue_in(sl, sl)
    for sl in (0, 1):
        wait_in(sl)
        compute(sl)
        issue_out(sl, sl)
        issue_in(sl, sl + 2)

    @pl.loop(2, NCHUNK - 2, step=2)
    def _(i):
        for sl in (0, 1):
            ch = i + sl
            wait_out(sl, ch - 2)
            wait_in(sl)
            compute(sl)
            issue_out(sl, ch)
            issue_in(sl, ch + 2)

    for sl in (0, 1):
        ch = NCHUNK - 2 + sl
        wait_out(sl, ch - 2)
        wait_in(sl)
        compute(sl)
        issue_out(sl, ch)
    for sl in (0, 1):
        wait_out(sl, NCHUNK - 2 + sl)


@functools.partial(
    pl.kernel,
    mesh=_mesh,
    out_type=jax.ShapeDtypeStruct((NC, NP, H), jnp.float32),
    scratch_types=[
        [pltpu.VMEM((K,), jnp.int32)] * 2,
        [pltpu.VMEM((K, H), jnp.float32)] * 2,
        pltpu.VMEM_SHARED((NP, H), jnp.float32),
        [pltpu.SemaphoreType.DMA] * 2,
        [pltpu.SemaphoreType.DMA] * 2,
    ],
)
def _sc_scatter(e_h, dst_h,
                msg_h,
                idx_d, ge, acc, esem, ssem):
    c, s, base = _worker_base()

    _acc_init(acc, ge[0], K, s)
    plsc.subcore_barrier()

    def issue_in(sl, ch):
        b0 = base + ch * K
        pltpu.sync_copy(dst_h.at[pl.ds(b0, K)], idx_d[sl])
        pltpu.async_copy(e_h.at[pl.ds(b0, K)], ge[sl], esem[sl])

    def wait_in(sl, ch):
        b0 = base + ch * K
        pltpu.make_async_copy(e_h.at[pl.ds(b0, K)], ge[sl], esem[sl]).wait()

    def issue_scatter(sl):
        pltpu.async_copy(ge[sl], acc.at[idx_d[sl]], ssem[sl], add=True)

    def wait_scatter(sl):
        pltpu.make_async_copy(ge[sl], acc.at[idx_d[sl]], ssem[sl]).wait()

    for sl in (0, 1):
        issue_in(sl, sl)
    for sl in (0, 1):
        wait_in(sl, sl)
        issue_scatter(sl)

    @pl.loop(2, NCHUNK, step=2)
    def _(i):
        for sl in (0, 1):
            ch = i + sl
            wait_scatter(sl)
            issue_in(sl, ch)
            wait_in(sl, ch)
            issue_scatter(sl)

    for sl in (0, 1):
        wait_scatter(sl)

    plsc.subcore_barrier()
    _acc_writeout(acc, msg_h, K, c, s)


@functools.partial(
    pl.kernel,
    mesh=_mesh,
    out_type=jax.ShapeDtypeStruct((E, H), jnp.float32),
    scratch_types=[
        [pltpu.VMEM((K,), jnp.int32)] * 2,
        [pltpu.VMEM((K,), jnp.int32)] * 2,
        [pltpu.VMEM((K, H), jnp.float32)] * 2,
        [pltpu.VMEM((K, H), jnp.float32)] * 2,
        [pltpu.VMEM((K, H), jnp.float32)] * 2,
        [pltpu.SemaphoreType.DMA] * 2,
        [pltpu.SemaphoreType.DMA] * 2,
    ],
)
def _sc_g2(a_h, b_h, src_h, dst_h,
           z3_h,
           idx_s, idx_d, ga, gb, oz, gsem, osem):
    c, s, base = _worker_base()

    def issue_in(sl, ch):
        b0 = base + ch * K
        pltpu.sync_copy(src_h.at[pl.ds(b0, K)], idx_s[sl])
        pltpu.sync_copy(dst_h.at[pl.ds(b0, K)], idx_d[sl])
        pltpu.async_copy(a_h.at[idx_s[sl]], ga[sl], gsem[sl])
        pltpu.async_copy(b_h.at[idx_d[sl]], gb[sl], gsem[sl])

    def wait_in(sl):
        pltpu.make_async_copy(a_h.at[idx_s[sl]], ga[sl], gsem[sl]).wait()
        pltpu.make_async_copy(b_h.at[idx_d[sl]], gb[sl], gsem[sl]).wait()

    def compute(sl):
        @pl.loop(0, K)
        def _(r):
            for cc in range(H // 16):
                o = 16 * cc
                va = ga[sl][r, pl.ds(o, 16)]
                vb = gb[sl][r, pl.ds(o, 16)]
                oz[sl][r, pl.ds(o, 16)] = va + vb

    def issue_out(sl, ch):
        b0 = base + ch * K
        pltpu.async_copy(oz[sl], z3_h.at[pl.ds(b0, K)], osem[sl])

    def wait_out(sl, ch):
        b0 = base + ch * K
        pltpu.make_async_copy(oz[sl], z3_h.at[pl.ds(b0, K)], osem[sl]).wait()

    for sl in (0, 1):
        issue_in(sl, sl)
    for sl in (0, 1):
        wait_in(sl)
        compute(sl)
        issue_out(sl, sl)
        issue_in(sl, sl + 2)

    @pl.loop(2, NCHUNK - 2, step=2)
    def _(i):
        for sl in (0, 1):
            ch = i + sl
            wait_out(sl, ch - 2)
            wait_in(sl)
            compute(sl)
            issue_out(sl, ch)
            issue_in(sl, ch + 2)

    for sl in (0, 1):
        ch = NCHUNK - 2 + sl
        wait_out(sl, ch - 2)
        wait_in(sl)
        compute(sl)
        issue_out(sl, ch)
    for sl in (0, 1):
        wait_out(sl, NCHUNK - 2 + sl)


KQ = 2000
NQCH = EPW // KQ


@functools.partial(
    pl.kernel,
    mesh=_mesh,
    out_type=jax.ShapeDtypeStruct((E,), jnp.float32),
    compiler_params=pltpu.CompilerParams(needs_layout_passes=False),
    scratch_types=[
        pltpu.VMEM((N,), jnp.float32),
        pltpu.VMEM((N,), jnp.float32),
        pltpu.VMEM((16,), jnp.float32),
        pltpu.VMEM((KQ,), jnp.int32),
        pltpu.VMEM((KQ,), jnp.int32),
        pltpu.VMEM((KQ,), jnp.float32),
    ],
)
def _sc_passq(ps_h, pd_h, src_h, dst_h, out0_h, bpe_h,
              q_h,
              pst, pdt, bpev, idx_s, idx_d, qb):
    c, s, base = _worker_base()

    pltpu.sync_copy(ps_h, pst)
    pltpu.sync_copy(pd_h, pdt)
    pltpu.sync_copy(bpe_h, bpev)

    @pl.loop(0, NQCH)
    def _(i):
        b0 = base + i * KQ
        pltpu.sync_copy(src_h.at[pl.ds(b0, KQ)], idx_s)
        pltpu.sync_copy(dst_h.at[pl.ds(b0, KQ)], idx_d)
        pltpu.sync_copy(out0_h.at[pl.ds(b0, KQ)], qb)

        @pl.loop(0, KQ // 16)
        def _(j):
            vs = idx_s[pl.ds(16 * j, 16)]
            vd = idx_d[pl.ds(16 * j, 16)]
            va = plsc.load_gather(pst, [vs])
            vb = plsc.load_gather(pdt, [vd])
            vo = qb[pl.ds(16 * j, 16)]
            qb[pl.ds(16 * j, 16)] = va + vb + vo + bpev[pl.ds(0, 16)]

        pltpu.sync_copy(qb, q_h.at[pl.ds(b0, KQ)])


def _tc_node_pre(nodes, W_ne, b_ne, W_ee, W_en):
    def body(nodes_ref, wne_ref, bne_ref, wee_ref, wen_ref,
             n0_ref, nsd_ref, ndb_ref):
        n0 = jnp.maximum(
            jnp.dot(nodes_ref[...], wne_ref[...],
                    preferred_element_type=jnp.float32) + bne_ref[...], 0.0)
        n0_ref[...] = n0
        nsd_ref[:, 0:H] = jnp.dot(n0, wee_ref[0:H, :],
                                  preferred_element_type=jnp.float32)
        nsd_ref[:, H:2 * H] = jnp.dot(n0, wen_ref[0:H, :],
                                      preferred_element_type=jnp.float32)
        ndb_ref[:, 0:H] = jnp.dot(n0, wee_ref[H:2 * H, :],
                                  preferred_element_type=jnp.float32)
        ndb_ref[:, H:2 * H] = jnp.dot(n0, wen_ref[H:2 * H, :],
                                      preferred_element_type=jnp.float32)

    return pl.pallas_call(
        body,
        out_shape=[
            jax.ShapeDtypeStruct((N, H), jnp.float32),
            jax.ShapeDtypeStruct((N, 2 * H), jnp.float32),
            jax.ShapeDtypeStruct((N, 2 * H), jnp.float32),
        ],
    )(nodes, W_ne, b_ne.reshape(1, H), W_ee, W_en)


def _tc_node_update(n, msgP, W_nn, b_nn, W_en):
    def body(n_ref, msg_ref, wnn_ref, bnn_ref, wen_ref,
             n1_ref, a_ref, b_ref):
        msg = msg_ref[0, 0:N, :] + msg_ref[1, 0:N, :]
        h = jnp.maximum(
            jnp.dot(n_ref[...], wnn_ref[0:H, :],
                    preferred_element_type=jnp.float32)
            + jnp.dot(msg, wnn_ref[H:2 * H, :],
                      preferred_element_type=jnp.float32)
            + bnn_ref[...], 0.0) + n_ref[...]
        n1_ref[...] = h
        a_ref[...] = jnp.dot(h, wen_ref[0:H, :],
                             preferred_element_type=jnp.float32)
        b_ref[...] = jnp.dot(h, wen_ref[H:2 * H, :],
                             preferred_element_type=jnp.float32)

    shp = jax.ShapeDtypeStruct((N, H), jnp.float32)
    return pl.pallas_call(
        body,
        out_shape=[shp, shp, shp],
    )(n, msgP, W_nn, b_nn.reshape(1, H), W_en)


def _tc_node_update2(n, msgP, W_nn, b_nn, Wpe_sd):
    def body(n_ref, msg_ref, wnn_ref, bnn_ref, wpe_ref, pspd_ref):
        msg = msg_ref[0, 0:N, :] + msg_ref[1, 0:N, :]
        h = jnp.maximum(
            jnp.dot(n_ref[...], wnn_ref[0:H, :],
                    preferred_element_type=jnp.float32)
            + jnp.dot(msg, wnn_ref[H:2 * H, :],
                      preferred_element_type=jnp.float32)
            + bnn_ref[...], 0.0) + n_ref[...]
        pspd_ref[...] = jnp.dot(h, wpe_ref[...],
                                preferred_element_type=jnp.float32)

    return pl.pallas_call(
        body,
        out_shape=jax.ShapeDtypeStruct((N, 8), jnp.float32),
    )(n, msgP, W_nn, b_nn.reshape(1, H), Wpe_sd)


BR = 2000


def _tc_edge_combine(e, z, W, b):
    def body(e_ref, z_ref, w_ref, b_ref, o_ref):
        o_ref[...] = jnp.maximum(
            z_ref[...] + jnp.dot(e_ref[...], w_ref[...],
                                 preferred_element_type=jnp.float32)
            + b_ref[...], 0.0) + e_ref[...]

    return pl.pallas_call(
        body,
        grid=(E // BR,),
        in_specs=[
            pl.BlockSpec((BR, H), lambda i: (i, 0)),
            pl.BlockSpec((BR, H), lambda i: (i, 0)),
            pl.BlockSpec((H, H), lambda i: (0, 0)),
            pl.BlockSpec((1, H), lambda i: (0, 0)),
        ],
        out_specs=pl.BlockSpec((BR, H), lambda i: (i, 0)),
        out_shape=jax.ShapeDtypeStruct((E, H), jnp.float32),
    )(e, z, W, b.reshape(1, H))


def _tc_final(e, z, W, b, wpe):
    def body(e_ref, z_ref, w_ref, b_ref, wpe_ref, o_ref):
        e3 = jnp.maximum(
            z_ref[...] + jnp.dot(e_ref[...], w_ref[...],
                                 preferred_element_type=jnp.float32)
            + b_ref[...], 0.0) + e_ref[...]
        o_ref[...] = jnp.sum(e3 * wpe_ref[...], axis=1, keepdims=True)

    return pl.pallas_call(
        body,
        grid=(E // BR,),
        in_specs=[
            pl.BlockSpec((BR, H), lambda i: (i, 0)),
            pl.BlockSpec((BR, H), lambda i: (i, 0)),
            pl.BlockSpec((H, H), lambda i: (0, 0)),
            pl.BlockSpec((1, H), lambda i: (0, 0)),
            pl.BlockSpec((1, H), lambda i: (0, 0)),
        ],
        out_specs=pl.BlockSpec((BR, 1), lambda i: (i, 0)),
        out_shape=jax.ShapeDtypeStruct((E, 1), jnp.float32),
    )(e, z, W, b.reshape(1, H), wpe.reshape(1, H))


def kernel(nodes, start_index, end_index, W_ne, b_ne, W_ee, b_ee,
           W_nn, b_nn, W_en, b_en, W_pe, b_pe):
    src = start_index.astype(jnp.int32)
    dst = end_index.astype(jnp.int32)

    n0, nsd, ndb = _tc_node_pre(nodes, W_ne, b_ne, W_ee, W_en)
    e1, z2 = _sc_g1(nsd, ndb, src, dst, b_ee)
    msg1 = _sc_scatter(e1, dst)
    e2 = _tc_edge_combine(e1, z2, W_en[2 * H:], b_en)
    n1, A1, B1 = _tc_node_update(n0, msg1, W_nn, b_nn, W_en)
    z3 = _sc_g2(A1, B1, src, dst)
    msg2 = _sc_scatter(e2, dst)
    out0 = _tc_final(e2, z3, W_en[2 * H:], b_en, W_pe[2 * H:, 0])
    Wpe_sd = jnp.concatenate(
        [W_pe[0:H], W_pe[H:2 * H], jnp.zeros((H, 6), jnp.float32)], axis=1)
    pspd = _tc_node_update2(n1, msg2, W_nn, b_nn, Wpe_sd)
    bpe16 = jnp.broadcast_to(b_pe, (16,))
    out = _sc_passq(pspd[:, 0], pspd[:, 1], src, dst, out0[:, 0], bpe16)
    return out


# speedup vs baseline: 1.2558x; 1.2558x over previous
import functools

import jax
import jax.numpy as jnp
from jax import lax
from jax.experimental import pallas as pl
from jax.experimental.pallas import tpu as pltpu
from jax.experimental.pallas import tpu_sc as plsc

N = 10000
E = 320000
H = 128

NC = 2
NS = 16
NW = NC * NS
EPW = E // NW
K = 40
NCHUNK = EPW // K
NP = 10240
RPS = NP // NS

_mesh = plsc.VectorSubcoreMesh(core_axis_name="c", subcore_axis_name="s")


def _worker_base():
    c = lax.axis_index("c")
    s = lax.axis_index("s")
    return c, s, (s * NC + c) * EPW


def _zero_rows(buf, rows):
    zv = jnp.zeros((16,), jnp.float32)

    @pl.loop(0, rows)
    def _(r):
        for c in range(H // 16):
            buf[r, pl.ds(16 * c, 16)] = zv


def _acc_init(acc, zbuf, rows, s):
    _zero_rows(zbuf, rows)
    base = s * RPS
    for j in range(RPS // rows):
        pltpu.sync_copy(zbuf, acc.at[pl.ds(base + j * rows, rows)])
    rem = RPS % rows
    if rem:
        pltpu.sync_copy(zbuf.at[pl.ds(0, rem)],
                        acc.at[pl.ds(base + (RPS // rows) * rows, rem)])


def _acc_writeout(acc, msg_hbm, rows, c, s):
    base = s * RPS
    for j in range(RPS // rows):
        pltpu.sync_copy(acc.at[pl.ds(base + j * rows, rows)],
                        msg_hbm.at[c, pl.ds(base + j * rows, rows)])
    rem = RPS % rows
    if rem:
        pltpu.sync_copy(acc.at[pl.ds(base + (RPS // rows) * rows, rem)],
                        msg_hbm.at[c, pl.ds(base + (RPS // rows) * rows, rem)])


@functools.partial(
    pl.kernel,
    mesh=_mesh,
    out_type=[
        jax.ShapeDtypeStruct((E, H), jnp.float32),
        jax.ShapeDtypeStruct((E, H), jnp.float32),
    ],
    scratch_types=[
        pltpu.VMEM((EPW,), jnp.int32),
        pltpu.VMEM((EPW,), jnp.int32),
        [pltpu.VMEM((K, 2 * H), jnp.float32)] * 2,
        [pltpu.VMEM((K, 2 * H), jnp.float32)] * 2,
        [pltpu.VMEM((K, H), jnp.float32)] * 2,
        [pltpu.VMEM((K, H), jnp.float32)] * 2,
        pltpu.VMEM((H,), jnp.float32),
        [pltpu.SemaphoreType.DMA] * 2,
        [pltpu.SemaphoreType.DMA] * 2,
    ],
)
def _sc_g1(nsd_h, ndb_h, src_h, dst_h, bee_h,
           e1_h, z2_h,
           idx_s, idx_d, gin_s, gin_d, oe, oz, bias_v, gsem, osem):
    c, s, base = _worker_base()

    pltpu.sync_copy(bee_h, bias_v)
    pltpu.sync_copy(src_h.at[pl.ds(base, EPW)], idx_s)
    pltpu.sync_copy(dst_h.at[pl.ds(base, EPW)], idx_d)

    def issue_in(sl, ch):
        o0 = ch * K
        pltpu.async_copy(nsd_h.at[idx_s.at[pl.ds(o0, K)]], gin_s[sl],
                         gsem[sl])
        pltpu.async_copy(ndb_h.at[idx_d.at[pl.ds(o0, K)]], gin_d[sl],
                         gsem[sl])

    def wait_in(sl, ch):
        o0 = ch * K
        pltpu.make_async_copy(nsd_h.at[idx_s.at[pl.ds(o0, K)]], gin_s[sl],
                              gsem[sl]).wait()
        pltpu.make_async_copy(ndb_h.at[idx_d.at[pl.ds(o0, K)]], gin_d[sl],
                              gsem[sl]).wait()

    def compute(sl):
        @pl.loop(0, K)
        def _(r):
            for cc in range(H // 16):
                o = 16 * cc
                vs = gin_s[sl][r, pl.ds(o, 16)]
                vd = gin_d[sl][r, pl.ds(o, 16)]
                bv = bias_v[pl.ds(o, 16)]
                oe[sl][r, pl.ds(o, 16)] = jnp.maximum(vs + vd + bv, 0.0)
                va = gin_s[sl][r, pl.ds(H + o, 16)]
                vb = gin_d[sl][r, pl.ds(H + o, 16)]
                oz[sl][r, pl.ds(o, 16)] = va + vb

    def issue_out(sl, ch):
        b0 = base + ch * K
        pltpu.async_copy(oe[sl], e1_h.at[pl.ds(b0, K)], osem[sl])
        pltpu.async_copy(oz[sl], z2_h.at[pl.ds(b0, K)], osem[sl])

    def wait_out(sl, ch):
        b0 = base + ch * K
        pltpu.make_async_copy(oe[sl], e1_h.at[pl.ds(b0, K)], osem[sl]).wait()
        pltpu.make_async_copy(oz[sl], z2_h.at[pl.ds(b0, K)], osem[sl]).wait()

    for sl in (0, 1):
        issue_in(sl, sl)
    for sl in (0, 1):
        wait_in(sl, sl)
        compute(sl)
        issue_out(sl, sl)
        issue_in(sl, sl + 2)

    @pl.loop(2, NCHUNK - 2, step=2)
    def _(i):
        for sl in (0, 1):
            ch = i + sl
            wait_out(sl, ch - 2)
            wait_in(sl, ch)
            compute(sl)
            issue_out(sl, ch)
            issue_in(sl, ch + 2)

    for sl in (0, 1):
        ch = NCHUNK - 2 + sl
        wait_out(sl, ch - 2)
        wait_in(sl, ch)
        compute(sl)
        issue_out(sl, ch)
    for sl in (0, 1):
        wait_out(sl, NCHUNK - 2 + sl)


@functools.partial(
    pl.kernel,
    mesh=_mesh,
    out_type=jax.ShapeDtypeStruct((NC, NP, H), jnp.float32),
    scratch_types=[
        pltpu.VMEM((NCHUNK, K), jnp.int32),
        [pltpu.VMEM((K, H), jnp.float32)] * 2,
        pltpu.VMEM_SHARED((NP, H), jnp.float32),
        [pltpu.SemaphoreType.DMA] * 2,
        [pltpu.SemaphoreType.DMA] * 2,
    ],
)
def _sc_scatter(e_h, dst3_h,
                msg_h,
                idx2, ge, acc, esem, ssem):
    c, s, base = _worker_base()
    wid = s * NC + c

    pltpu.sync_copy(dst3_h.at[wid], idx2)

    _acc_init(acc, ge[0], K, s)
    plsc.subcore_barrier()

    def issue_in(sl, ch):
        b0 = base + ch * K
        pltpu.async_copy(e_h.at[pl.ds(b0, K)], ge[sl], esem[sl])

    def wait_in(sl, ch):
        b0 = base + ch * K
        pltpu.make_async_copy(e_h.at[pl.ds(b0, K)], ge[sl], esem[sl]).wait()

    def issue_scatter(sl, ch):
        pltpu.async_copy(ge[sl], acc.at[idx2.at[ch]], ssem[sl], add=True)

    def wait_scatter(sl, ch):
        pltpu.make_async_copy(ge[sl], acc.at[idx2.at[ch]], ssem[sl]).wait()

    for sl in (0, 1):
        issue_in(sl, sl)
    for sl in (0, 1):
        wait_in(sl, sl)
        issue_scatter(sl, sl)

    @pl.loop(2, NCHUNK, step=2)
    def _(i):
        for sl in (0, 1):
            ch = i + sl
            wait_scatter(sl, ch - 2)
            issue_in(sl, ch)
            wait_in(sl, ch)
            issue_scatter(sl, ch)

    for sl in (0, 1):
        wait_scatter(sl, NCHUNK - 2 + sl)

    plsc.subcore_barrier()
    _acc_writeout(acc, msg_h, K, c, s)


@functools.partial(
    pl.kernel,
    mesh=_mesh,
    out_type=jax.ShapeDtypeStruct((E, H), jnp.float32),
    scratch_types=[
        pltpu.VMEM((EPW,), jnp.int32),
        pltpu.VMEM((EPW,), jnp.int32),
        [pltpu.VMEM((K, H), jnp.float32)] * 2,
        [pltpu.VMEM((K, H), jnp.float32)] * 2,
        [pltpu.VMEM((K, H), jnp.float32)] * 2,
        [pltpu.SemaphoreType.DMA] * 2,
        [pltpu.SemaphoreType.DMA] * 2,
    ],
)
def _sc_g2(a_h, b_h, src_h, dst_h,
           z3_h,
           idx_s, idx_d, ga, gb, oz, gsem, osem):
    c, s, base = _worker_base()

    pltpu.sync_copy(src_h.at[pl.ds(base, EPW)], idx_s)
    pltpu.sync_copy(dst_h.at[pl.ds(base, EPW)], idx_d)

    def issue_in(sl, ch):
        o0 = ch * K
        pltpu.async_copy(a_h.at[idx_s.at[pl.ds(o0, K)]], ga[sl], gsem[sl])
        pltpu.async_copy(b_h.at[idx_d.at[pl.ds(o0, K)]], gb[sl], gsem[sl])

    def wait_in(sl, ch):
        o0 = ch * K
        pltpu.make_async_copy(a_h.at[idx_s.at[pl.ds(o0, K)]], ga[sl],
                              gsem[sl]).wait()
        pltpu.make_async_copy(b_h.at[idx_d.at[pl.ds(o0, K)]], gb[sl],
                              gsem[sl]).wait()

    def compute(sl):
        @pl.loop(0, K)
        def _(r):
            for cc in range(H // 16):
                o = 16 * cc
                va = ga[sl][r, pl.ds(o, 16)]
                vb = gb[sl][r, pl.ds(o, 16)]
                oz[sl][r, pl.ds(o, 16)] = va + vb

    def issue_out(sl, ch):
        b0 = base + ch * K
        pltpu.async_copy(oz[sl], z3_h.at[pl.ds(b0, K)], osem[sl])

    def wait_out(sl, ch):
        b0 = base + ch * K
        pltpu.make_async_copy(oz[sl], z3_h.at[pl.ds(b0, K)], osem[sl]).wait()

    for sl in (0, 1):
        issue_in(sl, sl)
    for sl in (0, 1):
        wait_in(sl, sl)
        compute(sl)
        issue_out(sl, sl)
        issue_in(sl, sl + 2)

    @pl.loop(2, NCHUNK - 2, step=2)
    def _(i):
        for sl in (0, 1):
            ch = i + sl
            wait_out(sl, ch - 2)
            wait_in(sl, ch)
            compute(sl)
            issue_out(sl, ch)
            issue_in(sl, ch + 2)

    for sl in (0, 1):
        ch = NCHUNK - 2 + sl
        wait_out(sl, ch - 2)
        wait_in(sl, ch)
        compute(sl)
        issue_out(sl, ch)
    for sl in (0, 1):
        wait_out(sl, NCHUNK - 2 + sl)


KQ = 2000
NQCH = EPW // KQ


@functools.partial(
    pl.kernel,
    mesh=_mesh,
    out_type=jax.ShapeDtypeStruct((E,), jnp.float32),
    compiler_params=pltpu.CompilerParams(needs_layout_passes=False),
    scratch_types=[
        pltpu.VMEM((N,), jnp.float32),
        pltpu.VMEM((N,), jnp.float32),
        pltpu.VMEM((16,), jnp.float32),
        pltpu.VMEM((KQ,), jnp.int32),
        pltpu.VMEM((KQ,), jnp.int32),
        pltpu.VMEM((KQ,), jnp.float32),
    ],
)
def _sc_passq(ps_h, pd_h, src_h, dst_h, out0_h, bpe_h,
              q_h,
              pst, pdt, bpev, idx_s, idx_d, qb):
    c, s, base = _worker_base()

    pltpu.sync_copy(ps_h, pst)
    pltpu.sync_copy(pd_h, pdt)
    pltpu.sync_copy(bpe_h, bpev)

    @pl.loop(0, NQCH)
    def _(i):
        b0 = base + i * KQ
        pltpu.sync_copy(src_h.at[pl.ds(b0, KQ)], idx_s)
        pltpu.sync_copy(dst_h.at[pl.ds(b0, KQ)], idx_d)
        pltpu.sync_copy(out0_h.at[pl.ds(b0, KQ)], qb)

        @pl.loop(0, KQ // 16)
        def _(j):
            vs = idx_s[pl.ds(16 * j, 16)]
            vd = idx_d[pl.ds(16 * j, 16)]
            va = plsc.load_gather(pst, [vs])
            vb = plsc.load_gather(pdt, [vd])
            vo = qb[pl.ds(16 * j, 16)]
            qb[pl.ds(16 * j, 16)] = va + vb + vo + bpev[pl.ds(0, 16)]

        pltpu.sync_copy(qb, q_h.at[pl.ds(b0, KQ)])


def _tc_node_pre(nodes, W_ne, b_ne, W_ee, W_en):
    def body(nodes_ref, wne_ref, bne_ref, wee_ref, wen_ref,
             n0_ref, nsd_ref, ndb_ref):
        n0 = jnp.maximum(
            jnp.dot(nodes_ref[...], wne_ref[...],
                    preferred_element_type=jnp.float32) + bne_ref[...], 0.0)
        n0_ref[...] = n0
        nsd_ref[:, 0:H] = jnp.dot(n0, wee_ref[0:H, :],
                                  preferred_element_type=jnp.float32)
        nsd_ref[:, H:2 * H] = jnp.dot(n0, wen_ref[0:H, :],
                                      preferred_element_type=jnp.float32)
        ndb_ref[:, 0:H] = jnp.dot(n0, wee_ref[H:2 * H, :],
                                  preferred_element_type=jnp.float32)
        ndb_ref[:, H:2 * H] = jnp.dot(n0, wen_ref[H:2 * H, :],
                                      preferred_element_type=jnp.float32)

    return pl.pallas_call(
        body,
        out_shape=[
            jax.ShapeDtypeStruct((N, H), jnp.float32),
            jax.ShapeDtypeStruct((N, 2 * H), jnp.float32),
            jax.ShapeDtypeStruct((N, 2 * H), jnp.float32),
        ],
    )(nodes, W_ne, b_ne.reshape(1, H), W_ee, W_en)


def _tc_node_update(n, msgP, W_nn, b_nn, W_en):
    def body(n_ref, msg_ref, wnn_ref, bnn_ref, wen_ref,
             n1_ref, a_ref, b_ref):
        msg = msg_ref[0, 0:N, :] + msg_ref[1, 0:N, :]
        h = jnp.maximum(
            jnp.dot(n_ref[...], wnn_ref[0:H, :],
                    preferred_element_type=jnp.float32)
            + jnp.dot(msg, wnn_ref[H:2 * H, :],
                      preferred_element_type=jnp.float32)
            + bnn_ref[...], 0.0) + n_ref[...]
        n1_ref[...] = h
        a_ref[...] = jnp.dot(h, wen_ref[0:H, :],
                             preferred_element_type=jnp.float32)
        b_ref[...] = jnp.dot(h, wen_ref[H:2 * H, :],
                             preferred_element_type=jnp.float32)

    shp = jax.ShapeDtypeStruct((N, H), jnp.float32)
    return pl.pallas_call(
        body,
        out_shape=[shp, shp, shp],
    )(n, msgP, W_nn, b_nn.reshape(1, H), W_en)


def _tc_node_update2(n, msgP, W_nn, b_nn, Wpe_sd):
    def body(n_ref, msg_ref, wnn_ref, bnn_ref, wpe_ref, pspd_ref):
        msg = msg_ref[0, 0:N, :] + msg_ref[1, 0:N, :]
        h = jnp.maximum(
            jnp.dot(n_ref[...], wnn_ref[0:H, :],
                    preferred_element_type=jnp.float32)
            + jnp.dot(msg, wnn_ref[H:2 * H, :],
                      preferred_element_type=jnp.float32)
            + bnn_ref[...], 0.0) + n_ref[...]
        pspd_ref[...] = jnp.dot(h, wpe_ref[...],
                                preferred_element_type=jnp.float32)

    return pl.pallas_call(
        body,
        out_shape=jax.ShapeDtypeStruct((N, 8), jnp.float32),
    )(n, msgP, W_nn, b_nn.reshape(1, H), Wpe_sd)


BR = 2000


def _tc_edge_combine(e, z, W, b):
    def body(e_ref, z_ref, w_ref, b_ref, o_ref):
        o_ref[...] = jnp.maximum(
            z_ref[...] + jnp.dot(e_ref[...], w_ref[...],
                                 preferred_element_type=jnp.float32)
            + b_ref[...], 0.0) + e_ref[...]

    return pl.pallas_call(
        body,
        grid=(E // BR,),
        in_specs=[
            pl.BlockSpec((BR, H), lambda i: (i, 0)),
            pl.BlockSpec((BR, H), lambda i: (i, 0)),
            pl.BlockSpec((H, H), lambda i: (0, 0)),
            pl.BlockSpec((1, H), lambda i: (0, 0)),
        ],
        out_specs=pl.BlockSpec((BR, H), lambda i: (i, 0)),
        out_shape=jax.ShapeDtypeStruct((E, H), jnp.float32),
    )(e, z, W, b.reshape(1, H))


def _tc_final(e, z, W, b, wpe):
    def body(e_ref, z_ref, w_ref, b_ref, wpe_ref, o_ref):
        e3 = jnp.maximum(
            z_ref[...] + jnp.dot(e_ref[...], w_ref[...],
                                 preferred_element_type=jnp.float32)
            + b_ref[...], 0.0) + e_ref[...]
        o_ref[...] = jnp.sum(e3 * wpe_ref[...], axis=1, keepdims=True)

    return pl.pallas_call(
        body,
        grid=(E // BR,),
        in_specs=[
            pl.BlockSpec((BR, H), lambda i: (i, 0)),
            pl.BlockSpec((BR, H), lambda i: (i, 0)),
            pl.BlockSpec((H, H), lambda i: (0, 0)),
            pl.BlockSpec((1, H), lambda i: (0, 0)),
            pl.BlockSpec((1, H), lambda i: (0, 0)),
        ],
        out_specs=pl.BlockSpec((BR, 1), lambda i: (i, 0)),
        out_shape=jax.ShapeDtypeStruct((E, 1), jnp.float32),
    )(e, z, W, b.reshape(1, H), wpe.reshape(1, H))


def kernel(nodes, start_index, end_index, W_ne, b_ne, W_ee, b_ee,
           W_nn, b_nn, W_en, b_en, W_pe, b_pe):
    src = start_index.astype(jnp.int32)
    dst = end_index.astype(jnp.int32)

    n0, nsd, ndb = _tc_node_pre(nodes, W_ne, b_ne, W_ee, W_en)
    e1, z2 = _sc_g1(nsd, ndb, src, dst, b_ee)
    dst3 = dst.reshape(NW, NCHUNK, K)
    msg1 = _sc_scatter(e1, dst3)
    e2 = _tc_edge_combine(e1, z2, W_en[2 * H:], b_en)
    n1, A1, B1 = _tc_node_update(n0, msg1, W_nn, b_nn, W_en)
    z3 = _sc_g2(A1, B1, src, dst)
    msg2 = _sc_scatter(e2, dst3)
    out0 = _tc_final(e2, z3, W_en[2 * H:], b_en, W_pe[2 * H:, 0])
    Wpe_sd = jnp.concatenate(
        [W_pe[0:H], W_pe[H:2 * H], jnp.zeros((H, 6), jnp.float32)], axis=1)
    pspd = _tc_node_update2(n1, msg2, W_nn, b_nn, Wpe_sd)
    bpe16 = jnp.broadcast_to(b_pe, (16,))
    out = _sc_passq(pspd[:, 0], pspd[:, 1], src, dst, out0[:, 0], bpe16)
    return out


# speedup vs baseline: 1.4916x; 1.1878x over previous
import functools

import jax
import jax.numpy as jnp
from jax import lax
from jax.experimental import pallas as pl
from jax.experimental.pallas import tpu as pltpu
from jax.experimental.pallas import tpu_sc as plsc

N = 10000
E = 320000
H = 128

NC = 2
NS = 16
NW = NC * NS
EPW = E // NW
K = 40
NCHUNK = EPW // K
NP = 10240
RPS = NP // NS

_mesh = plsc.VectorSubcoreMesh(core_axis_name="c", subcore_axis_name="s")


def _worker_base():
    c = lax.axis_index("c")
    s = lax.axis_index("s")
    return c, s, (s * NC + c) * EPW


def _zero_rows(buf, rows):
    zv = jnp.zeros((16,), jnp.float32)

    @pl.loop(0, rows)
    def _(r):
        for c in range(H // 16):
            buf[r, pl.ds(16 * c, 16)] = zv


def _acc_init(acc, zbuf, rows, s):
    _zero_rows(zbuf, rows)
    base = s * RPS
    for j in range(RPS // rows):
        pltpu.sync_copy(zbuf, acc.at[pl.ds(base + j * rows, rows)])
    rem = RPS % rows
    if rem:
        pltpu.sync_copy(zbuf.at[pl.ds(0, rem)],
                        acc.at[pl.ds(base + (RPS // rows) * rows, rem)])


def _acc_writeout(acc, msg_hbm, rows, c, s):
    base = s * RPS
    for j in range(RPS // rows):
        pltpu.sync_copy(acc.at[pl.ds(base + j * rows, rows)],
                        msg_hbm.at[c, pl.ds(base + j * rows, rows)])
    rem = RPS % rows
    if rem:
        pltpu.sync_copy(acc.at[pl.ds(base + (RPS // rows) * rows, rem)],
                        msg_hbm.at[c, pl.ds(base + (RPS // rows) * rows, rem)])


def _make_gather_pass(with_bias):
    scratch = [
        pltpu.VMEM((EPW,), jnp.int32),
        pltpu.VMEM((EPW,), jnp.int32),
        [pltpu.VMEM((K, H), jnp.float32)] * 2,
        [pltpu.VMEM((K, H), jnp.float32)] * 2,
        [pltpu.VMEM((K, H), jnp.float32)] * 2,
        [pltpu.SemaphoreType.DMA] * 2,
        [pltpu.SemaphoreType.DMA] * 2,
    ]
    if with_bias:
        scratch.append(pltpu.VMEM((H,), jnp.float32))

    @functools.partial(
        pl.kernel,
        mesh=_mesh,
        out_type=jax.ShapeDtypeStruct((E, H), jnp.float32),
        scratch_types=scratch,
    )
    def gather_pass(a_h, b_h, src_h, dst_h, *rest):
        if with_bias:
            bee_h, o_h, idx_s, idx_d, ga, gb, oz, gsem, osem, bias_v = rest
        else:
            o_h, idx_s, idx_d, ga, gb, oz, gsem, osem = rest
        c, s, base = _worker_base()

        if with_bias:
            pltpu.sync_copy(bee_h, bias_v)
        pltpu.sync_copy(src_h.at[pl.ds(base, EPW)], idx_s)
        pltpu.sync_copy(dst_h.at[pl.ds(base, EPW)], idx_d)

        def issue_in(sl, ch):
            o0 = ch * K
            pltpu.async_copy(a_h.at[idx_s.at[pl.ds(o0, K)]], ga[sl],
                             gsem[sl])
            pltpu.async_copy(b_h.at[idx_d.at[pl.ds(o0, K)]], gb[sl],
                             gsem[sl])

        def wait_in(sl, ch):
            o0 = ch * K
            pltpu.make_async_copy(a_h.at[idx_s.at[pl.ds(o0, K)]], ga[sl],
                                  gsem[sl]).wait()
            pltpu.make_async_copy(b_h.at[idx_d.at[pl.ds(o0, K)]], gb[sl],
                                  gsem[sl]).wait()

        def compute(sl):
            @pl.loop(0, K)
            def _(r):
                for cc in range(H // 16):
                    o = 16 * cc
                    va = ga[sl][r, pl.ds(o, 16)]
                    vb = gb[sl][r, pl.ds(o, 16)]
                    if with_bias:
                        bv = bias_v[pl.ds(o, 16)]
                        oz[sl][r, pl.ds(o, 16)] = jnp.maximum(
                            va + vb + bv, 0.0)
                    else:
                        oz[sl][r, pl.ds(o, 16)] = va + vb

        def issue_out(sl, ch):
            b0 = base + ch * K
            pltpu.async_copy(oz[sl], o_h.at[pl.ds(b0, K)], osem[sl])

        def wait_out(sl, ch):
            b0 = base + ch * K
            pltpu.make_async_copy(oz[sl], o_h.at[pl.ds(b0, K)],
                                  osem[sl]).wait()

        for sl in (0, 1):
            issue_in(sl, sl)
        for sl in (0, 1):
            wait_in(sl, sl)
            compute(sl)
            issue_out(sl, sl)
            issue_in(sl, sl + 2)

        @pl.loop(2, NCHUNK - 2, step=2)
        def _(i):
            for sl in (0, 1):
                ch = i + sl
                wait_out(sl, ch - 2)
                wait_in(sl, ch)
                compute(sl)
                issue_out(sl, ch)
                issue_in(sl, ch + 2)

        for sl in (0, 1):
            ch = NCHUNK - 2 + sl
            wait_out(sl, ch - 2)
            wait_in(sl, ch)
            compute(sl)
            issue_out(sl, ch)
        for sl in (0, 1):
            wait_out(sl, NCHUNK - 2 + sl)

    return gather_pass


_sc_encode = _make_gather_pass(True)
_sc_gadd = _make_gather_pass(False)


@functools.partial(
    pl.kernel,
    mesh=_mesh,
    out_type=jax.ShapeDtypeStruct((NC, NP, H), jnp.float32),
    scratch_types=[
        pltpu.VMEM((NCHUNK, K), jnp.int32),
        [pltpu.VMEM((K, H), jnp.float32)] * 2,
        pltpu.VMEM_SHARED((NP, H), jnp.float32),
        [pltpu.SemaphoreType.DMA] * 2,
        [pltpu.SemaphoreType.DMA] * 2,
    ],
)
def _sc_scatter(e_h, dst3_h,
                msg_h,
                idx2, ge, acc, esem, ssem):
    c, s, base = _worker_base()
    wid = s * NC + c

    pltpu.sync_copy(dst3_h.at[wid], idx2)

    _acc_init(acc, ge[0], K, s)
    plsc.subcore_barrier()

    def issue_in(sl, ch):
        b0 = base + ch * K
        pltpu.async_copy(e_h.at[pl.ds(b0, K)], ge[sl], esem[sl])

    def wait_in(sl, ch):
        b0 = base + ch * K
        pltpu.make_async_copy(e_h.at[pl.ds(b0, K)], ge[sl], esem[sl]).wait()

    def issue_scatter(sl, ch):
        pltpu.async_copy(ge[sl], acc.at[idx2.at[ch]], ssem[sl], add=True)

    def wait_scatter(sl, ch):
        pltpu.make_async_copy(ge[sl], acc.at[idx2.at[ch]], ssem[sl]).wait()

    for sl in (0, 1):
        issue_in(sl, sl)
    for sl in (0, 1):
        wait_in(sl, sl)
        issue_scatter(sl, sl)

    @pl.loop(2, NCHUNK, step=2)
    def _(i):
        for sl in (0, 1):
            ch = i + sl
            wait_scatter(sl, ch - 2)
            issue_in(sl, ch)
            wait_in(sl, ch)
            issue_scatter(sl, ch)

    for sl in (0, 1):
        wait_scatter(sl, NCHUNK - 2 + sl)

    plsc.subcore_barrier()
    _acc_writeout(acc, msg_h, K, c, s)


KQ = 2000
NQCH = EPW // KQ


@functools.partial(
    pl.kernel,
    mesh=_mesh,
    out_type=jax.ShapeDtypeStruct((E,), jnp.float32),
    compiler_params=pltpu.CompilerParams(needs_layout_passes=False),
    scratch_types=[
        pltpu.VMEM((N,), jnp.float32),
        pltpu.VMEM((N,), jnp.float32),
        pltpu.VMEM((16,), jnp.float32),
        pltpu.VMEM((KQ,), jnp.int32),
        pltpu.VMEM((KQ,), jnp.int32),
        pltpu.VMEM((KQ,), jnp.float32),
    ],
)
def _sc_passq(ps_h, pd_h, src_h, dst_h, out0_h, bpe_h,
              q_h,
              pst, pdt, bpev, idx_s, idx_d, qb):
    c, s, base = _worker_base()

    pltpu.sync_copy(ps_h, pst)
    pltpu.sync_copy(pd_h, pdt)
    pltpu.sync_copy(bpe_h, bpev)

    @pl.loop(0, NQCH)
    def _(i):
        b0 = base + i * KQ
        pltpu.sync_copy(src_h.at[pl.ds(b0, KQ)], idx_s)
        pltpu.sync_copy(dst_h.at[pl.ds(b0, KQ)], idx_d)
        pltpu.sync_copy(out0_h.at[pl.ds(b0, KQ)], qb)

        @pl.loop(0, KQ // 16)
        def _(j):
            vs = idx_s[pl.ds(16 * j, 16)]
            vd = idx_d[pl.ds(16 * j, 16)]
            va = plsc.load_gather(pst, [vs])
            vb = plsc.load_gather(pdt, [vd])
            vo = qb[pl.ds(16 * j, 16)]
            qb[pl.ds(16 * j, 16)] = va + vb + vo + bpev[pl.ds(0, 16)]

        pltpu.sync_copy(qb, q_h.at[pl.ds(b0, KQ)])


def _tc_node_pre(nodes, W_ne, b_ne, W_ee, W_en):
    def body(nodes_ref, wne_ref, bne_ref, wee_ref, wen_ref,
             n0_ref, ns_ref, nd_ref, a_ref, b_ref):
        n0 = jnp.maximum(
            jnp.dot(nodes_ref[...], wne_ref[...],
                    preferred_element_type=jnp.float32) + bne_ref[...], 0.0)
        n0_ref[...] = n0
        ns_ref[...] = jnp.dot(n0, wee_ref[0:H, :],
                              preferred_element_type=jnp.float32)
        nd_ref[...] = jnp.dot(n0, wee_ref[H:2 * H, :],
                              preferred_element_type=jnp.float32)
        a_ref[...] = jnp.dot(n0, wen_ref[0:H, :],
                             preferred_element_type=jnp.float32)
        b_ref[...] = jnp.dot(n0, wen_ref[H:2 * H, :],
                             preferred_element_type=jnp.float32)

    shp = jax.ShapeDtypeStruct((N, H), jnp.float32)
    return pl.pallas_call(
        body,
        out_shape=[shp, shp, shp, shp, shp],
    )(nodes, W_ne, b_ne.reshape(1, H), W_ee, W_en)


def _tc_node_update(n, msgP, W_nn, b_nn, W_en):
    def body(n_ref, msg_ref, wnn_ref, bnn_ref, wen_ref,
             n1_ref, a_ref, b_ref):
        msg = msg_ref[0, 0:N, :] + msg_ref[1, 0:N, :]
        h = jnp.maximum(
            jnp.dot(n_ref[...], wnn_ref[0:H, :],
                    preferred_element_type=jnp.float32)
            + jnp.dot(msg, wnn_ref[H:2 * H, :],
                      preferred_element_type=jnp.float32)
            + bnn_ref[...], 0.0) + n_ref[...]
        n1_ref[...] = h
        a_ref[...] = jnp.dot(h, wen_ref[0:H, :],
                             preferred_element_type=jnp.float32)
        b_ref[...] = jnp.dot(h, wen_ref[H:2 * H, :],
                             preferred_element_type=jnp.float32)

    shp = jax.ShapeDtypeStruct((N, H), jnp.float32)
    return pl.pallas_call(
        body,
        out_shape=[shp, shp, shp],
    )(n, msgP, W_nn, b_nn.reshape(1, H), W_en)


def _tc_node_update2(n, msgP, W_nn, b_nn, Wpe_sd):
    def body(n_ref, msg_ref, wnn_ref, bnn_ref, wpe_ref, pspd_ref):
        msg = msg_ref[0, 0:N, :] + msg_ref[1, 0:N, :]
        h = jnp.maximum(
            jnp.dot(n_ref[...], wnn_ref[0:H, :],
                    preferred_element_type=jnp.float32)
            + jnp.dot(msg, wnn_ref[H:2 * H, :],
                      preferred_element_type=jnp.float32)
            + bnn_ref[...], 0.0) + n_ref[...]
        pspd_ref[...] = jnp.dot(h, wpe_ref[...],
                                preferred_element_type=jnp.float32)

    return pl.pallas_call(
        body,
        out_shape=jax.ShapeDtypeStruct((N, 8), jnp.float32),
    )(n, msgP, W_nn, b_nn.reshape(1, H), Wpe_sd)


BR = 2000


def _tc_edge_combine(e, z, W, b):
    def body(e_ref, z_ref, w_ref, b_ref, o_ref):
        o_ref[...] = jnp.maximum(
            z_ref[...] + jnp.dot(e_ref[...], w_ref[...],
                                 preferred_element_type=jnp.float32)
            + b_ref[...], 0.0) + e_ref[...]

    return pl.pallas_call(
        body,
        grid=(E // BR,),
        in_specs=[
            pl.BlockSpec((BR, H), lambda i: (i, 0)),
            pl.BlockSpec((BR, H), lambda i: (i, 0)),
            pl.BlockSpec((H, H), lambda i: (0, 0)),
            pl.BlockSpec((1, H), lambda i: (0, 0)),
        ],
        out_specs=pl.BlockSpec((BR, H), lambda i: (i, 0)),
        out_shape=jax.ShapeDtypeStruct((E, H), jnp.float32),
    )(e, z, W, b.reshape(1, H))


def _tc_final(e, z, W, b, wpe):
    def body(e_ref, z_ref, w_ref, b_ref, wpe_ref, o_ref):
        e3 = jnp.maximum(
            z_ref[...] + jnp.dot(e_ref[...], w_ref[...],
                                 preferred_element_type=jnp.float32)
            + b_ref[...], 0.0) + e_ref[...]
        o_ref[...] = jnp.sum(e3 * wpe_ref[...], axis=1, keepdims=True)

    return pl.pallas_call(
        body,
        grid=(E // BR,),
        in_specs=[
            pl.BlockSpec((BR, H), lambda i: (i, 0)),
            pl.BlockSpec((BR, H), lambda i: (i, 0)),
            pl.BlockSpec((H, H), lambda i: (0, 0)),
            pl.BlockSpec((1, H), lambda i: (0, 0)),
            pl.BlockSpec((1, H), lambda i: (0, 0)),
        ],
        out_specs=pl.BlockSpec((BR, 1), lambda i: (i, 0)),
        out_shape=jax.ShapeDtypeStruct((E, 1), jnp.float32),
    )(e, z, W, b.reshape(1, H), wpe.reshape(1, H))


def kernel(nodes, start_index, end_index, W_ne, b_ne, W_ee, b_ee,
           W_nn, b_nn, W_en, b_en, W_pe, b_pe):
    src = start_index.astype(jnp.int32)
    dst = end_index.astype(jnp.int32)

    n0, ns, nd, A0, B0 = _tc_node_pre(nodes, W_ne, b_ne, W_ee, W_en)
    e1 = _sc_encode(ns, nd, src, dst, b_ee)
    z2 = _sc_gadd(A0, B0, src, dst)
    dst3 = dst.reshape(NW, NCHUNK, K)
    msg1 = _sc_scatter(e1, dst3)
    e2 = _tc_edge_combine(e1, z2, W_en[2 * H:], b_en)
    n1, A1, B1 = _tc_node_update(n0, msg1, W_nn, b_nn, W_en)
    z3 = _sc_gadd(A1, B1, src, dst)
    msg2 = _sc_scatter(e2, dst3)
    out0 = _tc_final(e2, z3, W_en[2 * H:], b_en, W_pe[2 * H:, 0])
    Wpe_sd = jnp.concatenate(
        [W_pe[0:H], W_pe[H:2 * H], jnp.zeros((H, 6), jnp.float32)], axis=1)
    pspd = _tc_node_update2(n1, msg2, W_nn, b_nn, Wpe_sd)
    bpe16 = jnp.broadcast_to(b_pe, (16,))
    out = _sc_passq(pspd[:, 0], pspd[:, 1], src, dst, out0[:, 0], bpe16)
    return out


# speedup vs baseline: 1.6827x; 1.1281x over previous
import functools

import jax
import jax.numpy as jnp
from jax import lax
from jax.experimental import pallas as pl
from jax.experimental.pallas import tpu as pltpu
from jax.experimental.pallas import tpu_sc as plsc

N = 10000
E = 320000
H = 128

NC = 2
NS = 16
NW = NC * NS
EPW = E // NW
K = 40
NCHUNK = EPW // K
SLOTS = 5
NP = 10240
RPS = NP // NS

_mesh = plsc.VectorSubcoreMesh(core_axis_name="c", subcore_axis_name="s")


def _worker_base():
    c = lax.axis_index("c")
    s = lax.axis_index("s")
    return c, s, (s * NC + c) * EPW


def _zero_rows(buf, rows):
    zv = jnp.zeros((16,), jnp.float32)

    @pl.loop(0, rows)
    def _(r):
        for c in range(H // 16):
            buf[r, pl.ds(16 * c, 16)] = zv


def _acc_init(acc, zbuf, rows, s):
    _zero_rows(zbuf, rows)
    base = s * RPS
    for j in range(RPS // rows):
        pltpu.sync_copy(zbuf, acc.at[pl.ds(base + j * rows, rows)])
    rem = RPS % rows
    if rem:
        pltpu.sync_copy(zbuf.at[pl.ds(0, rem)],
                        acc.at[pl.ds(base + (RPS // rows) * rows, rem)])


def _acc_writeout(acc, msg_hbm, rows, c, s):
    base = s * RPS
    for j in range(RPS // rows):
        pltpu.sync_copy(acc.at[pl.ds(base + j * rows, rows)],
                        msg_hbm.at[c, pl.ds(base + j * rows, rows)])
    rem = RPS % rows
    if rem:
        pltpu.sync_copy(acc.at[pl.ds(base + (RPS // rows) * rows, rem)],
                        msg_hbm.at[c, pl.ds(base + (RPS // rows) * rows, rem)])


def _make_gather_pass(with_bias):
    scratch = [
        pltpu.VMEM((EPW,), jnp.int32),
        pltpu.VMEM((EPW,), jnp.int32),
        [pltpu.VMEM((K, H), jnp.float32)] * SLOTS,
        [pltpu.VMEM((K, H), jnp.float32)] * SLOTS,
        [pltpu.VMEM((K, H), jnp.float32)] * SLOTS,
        [pltpu.SemaphoreType.DMA] * SLOTS,
        [pltpu.SemaphoreType.DMA] * SLOTS,
    ]
    if with_bias:
        scratch.append(pltpu.VMEM((H,), jnp.float32))

    @functools.partial(
        pl.kernel,
        mesh=_mesh,
        out_type=jax.ShapeDtypeStruct((E, H), jnp.float32),
        scratch_types=scratch,
    )
    def gather_pass(a_h, b_h, src_h, dst_h, *rest):
        if with_bias:
            bee_h, o_h, idx_s, idx_d, ga, gb, oz, gsem, osem, bias_v = rest
        else:
            o_h, idx_s, idx_d, ga, gb, oz, gsem, osem = rest
        c, s, base = _worker_base()

        if with_bias:
            pltpu.sync_copy(bee_h, bias_v)
        pltpu.sync_copy(src_h.at[pl.ds(base, EPW)], idx_s)
        pltpu.sync_copy(dst_h.at[pl.ds(base, EPW)], idx_d)

        def issue_in(sl, ch):
            o0 = ch * K
            pltpu.async_copy(a_h.at[idx_s.at[pl.ds(o0, K)]], ga[sl],
                             gsem[sl])
            pltpu.async_copy(b_h.at[idx_d.at[pl.ds(o0, K)]], gb[sl],
                             gsem[sl])

        def wait_in(sl, ch):
            o0 = ch * K
            pltpu.make_async_copy(a_h.at[idx_s.at[pl.ds(o0, K)]], ga[sl],
                                  gsem[sl]).wait()
            pltpu.make_async_copy(b_h.at[idx_d.at[pl.ds(o0, K)]], gb[sl],
                                  gsem[sl]).wait()

        def compute(sl):
            @pl.loop(0, K)
            def _(r):
                for cc in range(H // 16):
                    o = 16 * cc
                    va = ga[sl][r, pl.ds(o, 16)]
                    vb = gb[sl][r, pl.ds(o, 16)]
                    if with_bias:
                        bv = bias_v[pl.ds(o, 16)]
                        oz[sl][r, pl.ds(o, 16)] = jnp.maximum(
                            va + vb + bv, 0.0)
                    else:
                        oz[sl][r, pl.ds(o, 16)] = va + vb

        def issue_out(sl, ch):
            b0 = base + ch * K
            pltpu.async_copy(oz[sl], o_h.at[pl.ds(b0, K)], osem[sl])

        def wait_out(sl, ch):
            b0 = base + ch * K
            pltpu.make_async_copy(oz[sl], o_h.at[pl.ds(b0, K)],
                                  osem[sl]).wait()

        slots = tuple(range(SLOTS))
        for sl in slots:
            issue_in(sl, sl)
        for sl in slots:
            wait_in(sl, sl)
            compute(sl)
            issue_out(sl, sl)
            issue_in(sl, sl + SLOTS)

        @pl.loop(SLOTS, NCHUNK - SLOTS, step=SLOTS)
        def _(i):
            for sl in slots:
                ch = i + sl
                wait_out(sl, ch - SLOTS)
                wait_in(sl, ch)
                compute(sl)
                issue_out(sl, ch)
                issue_in(sl, ch + SLOTS)

        for sl in slots:
            ch = NCHUNK - SLOTS + sl
            wait_out(sl, ch - SLOTS)
            wait_in(sl, ch)
            compute(sl)
            issue_out(sl, ch)
        for sl in slots:
            wait_out(sl, NCHUNK - SLOTS + sl)

    return gather_pass


_sc_encode = _make_gather_pass(True)
_sc_gadd = _make_gather_pass(False)


@functools.partial(
    pl.kernel,
    mesh=_mesh,
    out_type=jax.ShapeDtypeStruct((NC, NP, H), jnp.float32),
    scratch_types=[
        pltpu.VMEM((NCHUNK, K), jnp.int32),
        [pltpu.VMEM((K, H), jnp.float32)] * 2,
        pltpu.VMEM_SHARED((NP, H), jnp.float32),
        [pltpu.SemaphoreType.DMA] * 2,
        [pltpu.SemaphoreType.DMA] * 2,
    ],
)
def _sc_scatter(e_h, dst3_h,
                msg_h,
                idx2, ge, acc, esem, ssem):
    c, s, base = _worker_base()
    wid = s * NC + c

    pltpu.sync_copy(dst3_h.at[wid], idx2)

    _acc_init(acc, ge[0], K, s)
    plsc.subcore_barrier()

    def issue_in(sl, ch):
        b0 = base + ch * K
        pltpu.async_copy(e_h.at[pl.ds(b0, K)], ge[sl], esem[sl])

    def wait_in(sl, ch):
        b0 = base + ch * K
        pltpu.make_async_copy(e_h.at[pl.ds(b0, K)], ge[sl], esem[sl]).wait()

    def issue_scatter(sl, ch):
        pltpu.async_copy(ge[sl], acc.at[idx2.at[ch]], ssem[sl], add=True)

    def wait_scatter(sl, ch):
        pltpu.make_async_copy(ge[sl], acc.at[idx2.at[ch]], ssem[sl]).wait()

    for sl in (0, 1):
        issue_in(sl, sl)
    for sl in (0, 1):
        wait_in(sl, sl)
        issue_scatter(sl, sl)

    @pl.loop(2, NCHUNK, step=2)
    def _(i):
        for sl in (0, 1):
            wait_scatter(sl, i + sl - 2)
            issue_in(sl, i + sl)
        for sl in (0, 1):
            wait_in(sl, i + sl)
            issue_scatter(sl, i + sl)

    for sl in (0, 1):
        wait_scatter(sl, NCHUNK - 2 + sl)

    plsc.subcore_barrier()
    _acc_writeout(acc, msg_h, K, c, s)


KQ = 2000
NQCH = EPW // KQ


@functools.partial(
    pl.kernel,
    mesh=_mesh,
    out_type=jax.ShapeDtypeStruct((E,), jnp.float32),
    compiler_params=pltpu.CompilerParams(needs_layout_passes=False),
    scratch_types=[
        pltpu.VMEM((N,), jnp.float32),
        pltpu.VMEM((N,), jnp.float32),
        pltpu.VMEM((16,), jnp.float32),
        pltpu.VMEM((KQ,), jnp.int32),
        pltpu.VMEM((KQ,), jnp.int32),
        pltpu.VMEM((KQ,), jnp.float32),
    ],
)
def _sc_passq(ps_h, pd_h, src_h, dst_h, out0_h, bpe_h,
              q_h,
              pst, pdt, bpev, idx_s, idx_d, qb):
    c, s, base = _worker_base()

    pltpu.sync_copy(ps_h, pst)
    pltpu.sync_copy(pd_h, pdt)
    pltpu.sync_copy(bpe_h, bpev)

    @pl.loop(0, NQCH)
    def _(i):
        b0 = base + i * KQ
        pltpu.sync_copy(src_h.at[pl.ds(b0, KQ)], idx_s)
        pltpu.sync_copy(dst_h.at[pl.ds(b0, KQ)], idx_d)
        pltpu.sync_copy(out0_h.at[pl.ds(b0, KQ)], qb)

        @pl.loop(0, KQ // 16)
        def _(j):
            vs = idx_s[pl.ds(16 * j, 16)]
            vd = idx_d[pl.ds(16 * j, 16)]
            va = plsc.load_gather(pst, [vs])
            vb = plsc.load_gather(pdt, [vd])
            vo = qb[pl.ds(16 * j, 16)]
            qb[pl.ds(16 * j, 16)] = va + vb + vo + bpev[pl.ds(0, 16)]

        pltpu.sync_copy(qb, q_h.at[pl.ds(b0, KQ)])


def _tc_node_pre(nodes, W_ne, b_ne, W_ee, W_en):
    def body(nodes_ref, wne_ref, bne_ref, wee_ref, wen_ref,
             n0_ref, ns_ref, nd_ref, a_ref, b_ref):
        n0 = jnp.maximum(
            jnp.dot(nodes_ref[...], wne_ref[...],
                    preferred_element_type=jnp.float32) + bne_ref[...], 0.0)
        n0_ref[...] = n0
        ns_ref[...] = jnp.dot(n0, wee_ref[0:H, :],
                              preferred_element_type=jnp.float32)
        nd_ref[...] = jnp.dot(n0, wee_ref[H:2 * H, :],
                              preferred_element_type=jnp.float32)
        a_ref[...] = jnp.dot(n0, wen_ref[0:H, :],
                             preferred_element_type=jnp.float32)
        b_ref[...] = jnp.dot(n0, wen_ref[H:2 * H, :],
                             preferred_element_type=jnp.float32)

    shp = jax.ShapeDtypeStruct((N, H), jnp.float32)
    return pl.pallas_call(
        body,
        out_shape=[shp, shp, shp, shp, shp],
    )(nodes, W_ne, b_ne.reshape(1, H), W_ee, W_en)


def _tc_node_update(n, msgP, W_nn, b_nn, W_en):
    def body(n_ref, msg_ref, wnn_ref, bnn_ref, wen_ref,
             n1_ref, a_ref, b_ref):
        msg = msg_ref[0, 0:N, :] + msg_ref[1, 0:N, :]
        h = jnp.maximum(
            jnp.dot(n_ref[...], wnn_ref[0:H, :],
                    preferred_element_type=jnp.float32)
            + jnp.dot(msg, wnn_ref[H:2 * H, :],
                      preferred_element_type=jnp.float32)
            + bnn_ref[...], 0.0) + n_ref[...]
        n1_ref[...] = h
        a_ref[...] = jnp.dot(h, wen_ref[0:H, :],
                             preferred_element_type=jnp.float32)
        b_ref[...] = jnp.dot(h, wen_ref[H:2 * H, :],
                             preferred_element_type=jnp.float32)

    shp = jax.ShapeDtypeStruct((N, H), jnp.float32)
    return pl.pallas_call(
        body,
        out_shape=[shp, shp, shp],
    )(n, msgP, W_nn, b_nn.reshape(1, H), W_en)


def _tc_node_update2(n, msgP, W_nn, b_nn, Wpe_sd):
    def body(n_ref, msg_ref, wnn_ref, bnn_ref, wpe_ref, pspd_ref):
        msg = msg_ref[0, 0:N, :] + msg_ref[1, 0:N, :]
        h = jnp.maximum(
            jnp.dot(n_ref[...], wnn_ref[0:H, :],
                    preferred_element_type=jnp.float32)
            + jnp.dot(msg, wnn_ref[H:2 * H, :],
                      preferred_element_type=jnp.float32)
            + bnn_ref[...], 0.0) + n_ref[...]
        pspd_ref[...] = jnp.dot(h, wpe_ref[...],
                                preferred_element_type=jnp.float32)

    return pl.pallas_call(
        body,
        out_shape=jax.ShapeDtypeStruct((N, 8), jnp.float32),
    )(n, msgP, W_nn, b_nn.reshape(1, H), Wpe_sd)


BR = 2000


def _tc_edge_combine(e, z, W, b):
    def body(e_ref, z_ref, w_ref, b_ref, o_ref):
        o_ref[...] = jnp.maximum(
            z_ref[...] + jnp.dot(e_ref[...], w_ref[...],
                                 preferred_element_type=jnp.float32)
            + b_ref[...], 0.0) + e_ref[...]

    return pl.pallas_call(
        body,
        grid=(E // BR,),
        in_specs=[
            pl.BlockSpec((BR, H), lambda i: (i, 0)),
            pl.BlockSpec((BR, H), lambda i: (i, 0)),
            pl.BlockSpec((H, H), lambda i: (0, 0)),
            pl.BlockSpec((1, H), lambda i: (0, 0)),
        ],
        out_specs=pl.BlockSpec((BR, H), lambda i: (i, 0)),
        out_shape=jax.ShapeDtypeStruct((E, H), jnp.float32),
    )(e, z, W, b.reshape(1, H))


def _tc_final(e, z, W, b, wpe):
    def body(e_ref, z_ref, w_ref, b_ref, wpe_ref, o_ref):
        e3 = jnp.maximum(
            z_ref[...] + jnp.dot(e_ref[...], w_ref[...],
                                 preferred_element_type=jnp.float32)
            + b_ref[...], 0.0) + e_ref[...]
        o_ref[...] = jnp.sum(e3 * wpe_ref[...], axis=1, keepdims=True)

    return pl.pallas_call(
        body,
        grid=(E // BR,),
        in_specs=[
            pl.BlockSpec((BR, H), lambda i: (i, 0)),
            pl.BlockSpec((BR, H), lambda i: (i, 0)),
            pl.BlockSpec((H, H), lambda i: (0, 0)),
            pl.BlockSpec((1, H), lambda i: (0, 0)),
            pl.BlockSpec((1, H), lambda i: (0, 0)),
        ],
        out_specs=pl.BlockSpec((BR, 1), lambda i: (i, 0)),
        out_shape=jax.ShapeDtypeStruct((E, 1), jnp.float32),
    )(e, z, W, b.reshape(1, H), wpe.reshape(1, H))


def kernel(nodes, start_index, end_index, W_ne, b_ne, W_ee, b_ee,
           W_nn, b_nn, W_en, b_en, W_pe, b_pe):
    src = start_index.astype(jnp.int32)
    dst = end_index.astype(jnp.int32)

    n0, ns, nd, A0, B0 = _tc_node_pre(nodes, W_ne, b_ne, W_ee, W_en)
    e1 = _sc_encode(ns, nd, src, dst, b_ee)
    z2 = _sc_gadd(A0, B0, src, dst)
    dst3 = dst.reshape(NW, NCHUNK, K)
    msg1 = _sc_scatter(e1, dst3)
    e2 = _tc_edge_combine(e1, z2, W_en[2 * H:], b_en)
    n1, A1, B1 = _tc_node_update(n0, msg1, W_nn, b_nn, W_en)
    z3 = _sc_gadd(A1, B1, src, dst)
    msg2 = _sc_scatter(e2, dst3)
    out0 = _tc_final(e2, z3, W_en[2 * H:], b_en, W_pe[2 * H:, 0])
    Wpe_sd = jnp.concatenate(
        [W_pe[0:H], W_pe[H:2 * H], jnp.zeros((H, 6), jnp.float32)], axis=1)
    pspd = _tc_node_update2(n1, msg2, W_nn, b_nn, Wpe_sd)
    bpe16 = jnp.broadcast_to(b_pe, (16,))
    out = _sc_passq(pspd[:, 0], pspd[:, 1], src, dst, out0[:, 0], bpe16)
    return out


# speedup vs baseline: 2.0091x; 1.1940x over previous
import functools

import jax
import jax.numpy as jnp
from jax import lax
from jax.experimental import pallas as pl
from jax.experimental.pallas import tpu as pltpu
from jax.experimental.pallas import tpu_sc as plsc

N = 10000
E = 320000
H = 128

NC = 2
NS = 16
NW = NC * NS
EPW = E // NW
K = 40
NCHUNK = EPW // K
SLOTS = 5
NP = 10240
RPS = NP // NS

_mesh = plsc.VectorSubcoreMesh(core_axis_name="c", subcore_axis_name="s")


def _worker_base():
    c = lax.axis_index("c")
    s = lax.axis_index("s")
    return c, s, (s * NC + c) * EPW


def _zero_rows(buf, rows):
    zv = jnp.zeros((16,), jnp.float32)

    @pl.loop(0, rows)
    def _(r):
        for c in range(H // 16):
            buf[r, pl.ds(16 * c, 16)] = zv


def _acc_init(acc, zbuf, rows, s):
    _zero_rows(zbuf, rows)
    base = s * RPS
    for j in range(RPS // rows):
        pltpu.sync_copy(zbuf, acc.at[pl.ds(base + j * rows, rows)])
    rem = RPS % rows
    if rem:
        pltpu.sync_copy(zbuf.at[pl.ds(0, rem)],
                        acc.at[pl.ds(base + (RPS // rows) * rows, rem)])


def _acc_writeout(acc, msg_hbm, rows, c, s):
    base = s * RPS
    for j in range(RPS // rows):
        pltpu.sync_copy(acc.at[pl.ds(base + j * rows, rows)],
                        msg_hbm.at[c, pl.ds(base + j * rows, rows)])
    rem = RPS % rows
    if rem:
        pltpu.sync_copy(acc.at[pl.ds(base + (RPS // rows) * rows, rem)],
                        msg_hbm.at[c, pl.ds(base + (RPS // rows) * rows, rem)])


def _make_gather_pass(with_bias):
    scratch = [
        pltpu.VMEM((EPW,), jnp.int32),
        pltpu.VMEM((EPW,), jnp.int32),
        [pltpu.VMEM((K, H), jnp.float32)] * SLOTS,
        [pltpu.VMEM((K, H), jnp.float32)] * SLOTS,
        [pltpu.VMEM((K, H), jnp.float32)] * SLOTS,
        [pltpu.SemaphoreType.DMA] * SLOTS,
        [pltpu.SemaphoreType.DMA] * SLOTS,
    ]
    if with_bias:
        scratch.append(pltpu.VMEM((H,), jnp.float32))

    @functools.partial(
        pl.kernel,
        mesh=_mesh,
        out_type=jax.ShapeDtypeStruct((E, H), jnp.float32),
        scratch_types=scratch,
    )
    def gather_pass(a_h, b_h, src_h, dst_h, *rest):
        if with_bias:
            bee_h, o_h, idx_s, idx_d, ga, gb, oz, gsem, osem, bias_v = rest
        else:
            o_h, idx_s, idx_d, ga, gb, oz, gsem, osem = rest
        c, s, base = _worker_base()

        if with_bias:
            pltpu.sync_copy(bee_h, bias_v)
            bvs = [bias_v[pl.ds(16 * cc, 16)] for cc in range(H // 16)]
        pltpu.sync_copy(src_h.at[pl.ds(base, EPW)], idx_s)
        pltpu.sync_copy(dst_h.at[pl.ds(base, EPW)], idx_d)

        def issue_in(sl, ch):
            o0 = ch * K
            pltpu.async_copy(a_h.at[idx_s.at[pl.ds(o0, K)]], ga[sl],
                             gsem[sl])
            pltpu.async_copy(b_h.at[idx_d.at[pl.ds(o0, K)]], gb[sl],
                             gsem[sl])

        def wait_in(sl, ch):
            o0 = ch * K
            pltpu.make_async_copy(a_h.at[idx_s.at[pl.ds(o0, K)]], ga[sl],
                                  gsem[sl]).wait()
            pltpu.make_async_copy(b_h.at[idx_d.at[pl.ds(o0, K)]], gb[sl],
                                  gsem[sl]).wait()

        def compute(sl):
            @pl.loop(0, K)
            def _(r):
                for cc in range(H // 16):
                    o = 16 * cc
                    va = ga[sl][r, pl.ds(o, 16)]
                    vb = gb[sl][r, pl.ds(o, 16)]
                    if with_bias:
                        oz[sl][r, pl.ds(o, 16)] = jnp.maximum(
                            va + vb + bvs[cc], 0.0)
                    else:
                        oz[sl][r, pl.ds(o, 16)] = va + vb

        def issue_out(sl, ch):
            b0 = base + ch * K
            pltpu.async_copy(oz[sl], o_h.at[pl.ds(b0, K)], osem[sl])

        def wait_out(sl, ch):
            b0 = base + ch * K
            pltpu.make_async_copy(oz[sl], o_h.at[pl.ds(b0, K)],
                                  osem[sl]).wait()

        slots = tuple(range(SLOTS))
        for sl in slots:
            issue_in(sl, sl)
        for sl in slots:
            wait_in(sl, sl)
            compute(sl)
            issue_out(sl, sl)
            issue_in(sl, sl + SLOTS)

        @pl.loop(SLOTS, NCHUNK - SLOTS, step=SLOTS)
        def _(i):
            for sl in slots:
                ch = i + sl
                wait_out(sl, ch - SLOTS)
                wait_in(sl, ch)
                compute(sl)
                issue_out(sl, ch)
                issue_in(sl, ch + SLOTS)

        for sl in slots:
            ch = NCHUNK - SLOTS + sl
            wait_out(sl, ch - SLOTS)
            wait_in(sl, ch)
            compute(sl)
            issue_out(sl, ch)
        for sl in slots:
            wait_out(sl, NCHUNK - SLOTS + sl)

    return gather_pass


_sc_encode = _make_gather_pass(True)
_sc_gadd = _make_gather_pass(False)


@functools.partial(
    pl.kernel,
    mesh=_mesh,
    out_type=jax.ShapeDtypeStruct((NC, NP, H), jnp.float32),
    scratch_types=[
        pltpu.VMEM((NCHUNK, K), jnp.int32),
        [pltpu.VMEM((K, H), jnp.float32)] * 2,
        pltpu.VMEM_SHARED((NP, H), jnp.float32),
        [pltpu.SemaphoreType.DMA] * 2,
        [pltpu.SemaphoreType.DMA] * 2,
    ],
)
def _sc_scatter(e_h, dst3_h,
                msg_h,
                idx2, ge, acc, esem, ssem):
    c, s, base = _worker_base()
    wid = s * NC + c

    pltpu.sync_copy(dst3_h.at[wid], idx2)

    _acc_init(acc, ge[0], K, s)
    plsc.subcore_barrier()

    def issue_in(sl, ch):
        b0 = base + ch * K
        pltpu.async_copy(e_h.at[pl.ds(b0, K)], ge[sl], esem[sl])

    def wait_in(sl, ch):
        b0 = base + ch * K
        pltpu.make_async_copy(e_h.at[pl.ds(b0, K)], ge[sl], esem[sl]).wait()

    def issue_scatter(sl, ch):
        pltpu.async_copy(ge[sl], acc.at[idx2.at[ch]], ssem[sl], add=True)

    def wait_scatter(sl, ch):
        pltpu.make_async_copy(ge[sl], acc.at[idx2.at[ch]], ssem[sl]).wait()

    for sl in (0, 1):
        issue_in(sl, sl)
    for sl in (0, 1):
        wait_in(sl, sl)
        issue_scatter(sl, sl)

    @pl.loop(2, NCHUNK, step=2)
    def _(i):
        for sl in (0, 1):
            wait_scatter(sl, i + sl - 2)
            issue_in(sl, i + sl)
        for sl in (0, 1):
            wait_in(sl, i + sl)
            issue_scatter(sl, i + sl)

    for sl in (0, 1):
        wait_scatter(sl, NCHUNK - 2 + sl)

    plsc.subcore_barrier()
    _acc_writeout(acc, msg_h, K, c, s)


KQ = 2000
NQCH = EPW // KQ


@functools.partial(
    pl.kernel,
    mesh=_mesh,
    out_type=jax.ShapeDtypeStruct((E,), jnp.float32),
    compiler_params=pltpu.CompilerParams(needs_layout_passes=False),
    scratch_types=[
        pltpu.VMEM((N,), jnp.float32),
        pltpu.VMEM((N,), jnp.float32),
        pltpu.VMEM((16,), jnp.float32),
        pltpu.VMEM((KQ,), jnp.int32),
        pltpu.VMEM((KQ,), jnp.int32),
        pltpu.VMEM((KQ,), jnp.float32),
    ],
)
def _sc_passq(ps_h, pd_h, src_h, dst_h, out0_h, bpe_h,
              q_h,
              pst, pdt, bpev, idx_s, idx_d, qb):
    c, s, base = _worker_base()

    pltpu.sync_copy(ps_h, pst)
    pltpu.sync_copy(pd_h, pdt)
    pltpu.sync_copy(bpe_h, bpev)

    @pl.loop(0, NQCH)
    def _(i):
        b0 = base + i * KQ
        pltpu.sync_copy(src_h.at[pl.ds(b0, KQ)], idx_s)
        pltpu.sync_copy(dst_h.at[pl.ds(b0, KQ)], idx_d)
        pltpu.sync_copy(out0_h.at[pl.ds(b0, KQ)], qb)

        @pl.loop(0, KQ // 16)
        def _(j):
            vs = idx_s[pl.ds(16 * j, 16)]
            vd = idx_d[pl.ds(16 * j, 16)]
            va = plsc.load_gather(pst, [vs])
            vb = plsc.load_gather(pdt, [vd])
            vo = qb[pl.ds(16 * j, 16)]
            qb[pl.ds(16 * j, 16)] = va + vb + vo + bpev[pl.ds(0, 16)]

        pltpu.sync_copy(qb, q_h.at[pl.ds(b0, KQ)])


def _tc_node_pre(nodes, W_ne, b_ne, W_ee, W_en):
    def body(nodes_ref, wne_ref, bne_ref, wee_ref, wen_ref,
             n0_ref, ns_ref, nd_ref, a_ref, b_ref):
        n0 = jnp.maximum(
            jnp.dot(nodes_ref[...], wne_ref[...],
                    preferred_element_type=jnp.float32) + bne_ref[...], 0.0)
        n0_ref[...] = n0
        ns_ref[...] = jnp.dot(n0, wee_ref[0:H, :],
                              preferred_element_type=jnp.float32)
        nd_ref[...] = jnp.dot(n0, wee_ref[H:2 * H, :],
                              preferred_element_type=jnp.float32)
        a_ref[...] = jnp.dot(n0, wen_ref[0:H, :],
                             preferred_element_type=jnp.float32)
        b_ref[...] = jnp.dot(n0, wen_ref[H:2 * H, :],
                             preferred_element_type=jnp.float32)

    shp = jax.ShapeDtypeStruct((N, H), jnp.float32)
    return pl.pallas_call(
        body,
        out_shape=[shp, shp, shp, shp, shp],
    )(nodes, W_ne, b_ne.reshape(1, H), W_ee, W_en)


def _tc_node_update(n, msgP, W_nn, b_nn, W_en):
    def body(n_ref, msg_ref, wnn_ref, bnn_ref, wen_ref,
             n1_ref, a_ref, b_ref):
        msg = msg_ref[0, 0:N, :] + msg_ref[1, 0:N, :]
        h = jnp.maximum(
            jnp.dot(n_ref[...], wnn_ref[0:H, :],
                    preferred_element_type=jnp.float32)
            + jnp.dot(msg, wnn_ref[H:2 * H, :],
                      preferred_element_type=jnp.float32)
            + bnn_ref[...], 0.0) + n_ref[...]
        n1_ref[...] = h
        a_ref[...] = jnp.dot(h, wen_ref[0:H, :],
                             preferred_element_type=jnp.float32)
        b_ref[...] = jnp.dot(h, wen_ref[H:2 * H, :],
                             preferred_element_type=jnp.float32)

    shp = jax.ShapeDtypeStruct((N, H), jnp.float32)
    return pl.pallas_call(
        body,
        out_shape=[shp, shp, shp],
    )(n, msgP, W_nn, b_nn.reshape(1, H), W_en)


def _tc_node_update2(n, msgP, W_nn, b_nn, Wpe_sd):
    def body(n_ref, msg_ref, wnn_ref, bnn_ref, wpe_ref, pspd_ref):
        msg = msg_ref[0, 0:N, :] + msg_ref[1, 0:N, :]
        h = jnp.maximum(
            jnp.dot(n_ref[...], wnn_ref[0:H, :],
                    preferred_element_type=jnp.float32)
            + jnp.dot(msg, wnn_ref[H:2 * H, :],
                      preferred_element_type=jnp.float32)
            + bnn_ref[...], 0.0) + n_ref[...]
        pspd_ref[...] = jnp.dot(h, wpe_ref[...],
                                preferred_element_type=jnp.float32)

    return pl.pallas_call(
        body,
        out_shape=jax.ShapeDtypeStruct((N, 8), jnp.float32),
    )(n, msgP, W_nn, b_nn.reshape(1, H), Wpe_sd)


BR = 2000


def _tc_edge_combine(e, z, W, b):
    def body(e_ref, z_ref, w_ref, b_ref, o_ref):
        o_ref[...] = jnp.maximum(
            z_ref[...] + jnp.dot(e_ref[...], w_ref[...],
                                 preferred_element_type=jnp.float32)
            + b_ref[...], 0.0) + e_ref[...]

    return pl.pallas_call(
        body,
        grid=(E // BR,),
        in_specs=[
            pl.BlockSpec((BR, H), lambda i: (i, 0)),
            pl.BlockSpec((BR, H), lambda i: (i, 0)),
            pl.BlockSpec((H, H), lambda i: (0, 0)),
            pl.BlockSpec((1, H), lambda i: (0, 0)),
        ],
        out_specs=pl.BlockSpec((BR, H), lambda i: (i, 0)),
        out_shape=jax.ShapeDtypeStruct((E, H), jnp.float32),
    )(e, z, W, b.reshape(1, H))


def _tc_final(e, z, W, b, wpe):
    def body(e_ref, z_ref, w_ref, b_ref, wpe_ref, o_ref):
        e3 = jnp.maximum(
            z_ref[...] + jnp.dot(e_ref[...], w_ref[...],
                                 preferred_element_type=jnp.float32)
            + b_ref[...], 0.0) + e_ref[...]
        o_ref[...] = jnp.sum(e3 * wpe_ref[...], axis=1, keepdims=True)

    return pl.pallas_call(
        body,
        grid=(E // BR,),
        in_specs=[
            pl.BlockSpec((BR, H), lambda i: (i, 0)),
            pl.BlockSpec((BR, H), lambda i: (i, 0)),
            pl.BlockSpec((H, H), lambda i: (0, 0)),
            pl.BlockSpec((1, H), lambda i: (0, 0)),
            pl.BlockSpec((1, H), lambda i: (0, 0)),
        ],
        out_specs=pl.BlockSpec((BR, 1), lambda i: (i, 0)),
        out_shape=jax.ShapeDtypeStruct((E, 1), jnp.float32),
    )(e, z, W, b.reshape(1, H), wpe.reshape(1, H))


def kernel(nodes, start_index, end_index, W_ne, b_ne, W_ee, b_ee,
           W_nn, b_nn, W_en, b_en, W_pe, b_pe):
    src = start_index.astype(jnp.int32)
    dst = end_index.astype(jnp.int32)

    n0, ns, nd, A0, B0 = _tc_node_pre(nodes, W_ne, b_ne, W_ee, W_en)
    e1 = _sc_encode(ns, nd, src, dst, b_ee)
    z2 = _sc_gadd(A0, B0, src, dst)
    dst3 = dst.reshape(NW, NCHUNK, K)
    msg1 = _sc_scatter(e1, dst3)
    e2 = _tc_edge_combine(e1, z2, W_en[2 * H:], b_en)
    n1, A1, B1 = _tc_node_update(n0, msg1, W_nn, b_nn, W_en)
    z3 = _sc_gadd(A1, B1, src, dst)
    msg2 = _sc_scatter(e2, dst3)
    out0 = _tc_final(e2, z3, W_en[2 * H:], b_en, W_pe[2 * H:, 0])
    Wpe_sd = jnp.concatenate(
        [W_pe[0:H], W_pe[H:2 * H], jnp.zeros((H, 6), jnp.float32)], axis=1)
    pspd = _tc_node_update2(n1, msg2, W_nn, b_nn, Wpe_sd)
    bpe16 = jnp.broadcast_to(b_pe, (16,))
    out = _sc_passq(pspd[:, 0], pspd[:, 1], src, dst, out0[:, 0], bpe16)
    return out


# speedup vs baseline: 2.0354x; 1.0131x over previous
import functools

import jax
import jax.numpy as jnp
from jax import lax
from jax.experimental import pallas as pl
from jax.experimental.pallas import tpu as pltpu
from jax.experimental.pallas import tpu_sc as plsc

N = 10000
E = 320000
H = 128

NC = 2
NS = 16
NW = NC * NS
EPW = E // NW
K = 40
NCHUNK = EPW // K
SLOTS = 5
NP = 10240
RPS = NP // NS

_mesh = plsc.VectorSubcoreMesh(core_axis_name="c", subcore_axis_name="s")


def _worker_base():
    c = lax.axis_index("c")
    s = lax.axis_index("s")
    return c, s, (s * NC + c) * EPW


def _zero_rows(buf, rows):
    zv = jnp.zeros((16,), jnp.float32)

    @pl.loop(0, rows)
    def _(r):
        for c in range(H // 16):
            buf[r, pl.ds(16 * c, 16)] = zv


def _acc_init(acc, zbuf, rows, s):
    _zero_rows(zbuf, rows)
    base = s * RPS
    for j in range(RPS // rows):
        pltpu.sync_copy(zbuf, acc.at[pl.ds(base + j * rows, rows)])
    rem = RPS % rows
    if rem:
        pltpu.sync_copy(zbuf.at[pl.ds(0, rem)],
                        acc.at[pl.ds(base + (RPS // rows) * rows, rem)])


def _acc_writeout(acc, msg_hbm, rows, c, s):
    base = s * RPS
    for j in range(RPS // rows):
        pltpu.sync_copy(acc.at[pl.ds(base + j * rows, rows)],
                        msg_hbm.at[c, pl.ds(base + j * rows, rows)])
    rem = RPS % rows
    if rem:
        pltpu.sync_copy(acc.at[pl.ds(base + (RPS // rows) * rows, rem)],
                        msg_hbm.at[c, pl.ds(base + (RPS // rows) * rows, rem)])


def _make_gather_pass(with_bias, epw=EPW, nchunk=NCHUNK):
    scratch = [
        pltpu.VMEM((epw,), jnp.int32),
        pltpu.VMEM((epw,), jnp.int32),
        [pltpu.VMEM((K, H), jnp.float32)] * SLOTS,
        [pltpu.VMEM((K, H), jnp.float32)] * SLOTS,
        [pltpu.VMEM((K, H), jnp.float32)] * SLOTS,
        [pltpu.SemaphoreType.DMA] * SLOTS,
        [pltpu.SemaphoreType.DMA] * SLOTS,
    ]
    if with_bias:
        scratch.append(pltpu.VMEM((H,), jnp.float32))

    @functools.partial(
        pl.kernel,
        mesh=_mesh,
        out_type=jax.ShapeDtypeStruct((epw * NW, H), jnp.float32),
        scratch_types=scratch,
    )
    def gather_pass(a_h, b_h, src_h, dst_h, *rest):
        if with_bias:
            bee_h, o_h, idx_s, idx_d, ga, gb, oz, gsem, osem, bias_v = rest
        else:
            o_h, idx_s, idx_d, ga, gb, oz, gsem, osem = rest
        c = lax.axis_index("c")
        s = lax.axis_index("s")
        base = (s * NC + c) * epw

        if with_bias:
            pltpu.sync_copy(bee_h, bias_v)
            bvs = [bias_v[pl.ds(16 * cc, 16)] for cc in range(H // 16)]
        pltpu.sync_copy(src_h.at[pl.ds(base, epw)], idx_s)
        pltpu.sync_copy(dst_h.at[pl.ds(base, epw)], idx_d)

        def issue_in(sl, ch):
            o0 = ch * K
            pltpu.async_copy(a_h.at[idx_s.at[pl.ds(o0, K)]], ga[sl],
                             gsem[sl])
            pltpu.async_copy(b_h.at[idx_d.at[pl.ds(o0, K)]], gb[sl],
                             gsem[sl])

        def wait_in(sl, ch):
            o0 = ch * K
            pltpu.make_async_copy(a_h.at[idx_s.at[pl.ds(o0, K)]], ga[sl],
                                  gsem[sl]).wait()
            pltpu.make_async_copy(b_h.at[idx_d.at[pl.ds(o0, K)]], gb[sl],
                                  gsem[sl]).wait()

        def compute(sl):
            @pl.loop(0, K)
            def _(r):
                for cc in range(H // 16):
                    o = 16 * cc
                    va = ga[sl][r, pl.ds(o, 16)]
                    vb = gb[sl][r, pl.ds(o, 16)]
                    if with_bias:
                        oz[sl][r, pl.ds(o, 16)] = jnp.maximum(
                            va + vb + bvs[cc], 0.0)
                    else:
                        oz[sl][r, pl.ds(o, 16)] = va + vb

        def issue_out(sl, ch):
            b0 = base + ch * K
            pltpu.async_copy(oz[sl], o_h.at[pl.ds(b0, K)], osem[sl])

        def wait_out(sl, ch):
            b0 = base + ch * K
            pltpu.make_async_copy(oz[sl], o_h.at[pl.ds(b0, K)],
                                  osem[sl]).wait()

        slots = tuple(range(SLOTS))
        for sl in slots:
            issue_in(sl, sl)
        for sl in slots:
            wait_in(sl, sl)
            compute(sl)
            issue_out(sl, sl)
            issue_in(sl, sl + SLOTS)

        @pl.loop(SLOTS, nchunk - SLOTS, step=SLOTS)
        def _(i):
            for sl in slots:
                ch = i + sl
                wait_out(sl, ch - SLOTS)
                wait_in(sl, ch)
                compute(sl)
                issue_out(sl, ch)
                issue_in(sl, ch + SLOTS)

        for sl in slots:
            ch = nchunk - SLOTS + sl
            wait_out(sl, ch - SLOTS)
            wait_in(sl, ch)
            compute(sl)
            issue_out(sl, ch)
        for sl in slots:
            wait_out(sl, nchunk - SLOTS + sl)

    return gather_pass


EH = E // 2
EPW2 = EH // NW
NCHUNK2 = EPW2 // K

_sc_encode = _make_gather_pass(True)
_sc_gadd = _make_gather_pass(False)
_sc_gadd_half = _make_gather_pass(False, EPW2, NCHUNK2)


@functools.partial(
    pl.kernel,
    mesh=_mesh,
    out_type=jax.ShapeDtypeStruct((NC, NP, H), jnp.float32),
    scratch_types=[
        pltpu.VMEM((NCHUNK, K), jnp.int32),
        [pltpu.VMEM((K, H), jnp.float32)] * 2,
        pltpu.VMEM_SHARED((NP, H), jnp.float32),
        [pltpu.SemaphoreType.DMA] * 2,
        [pltpu.SemaphoreType.DMA] * 2,
    ],
)
def _sc_scatter(e_h, dst3_h,
                msg_h,
                idx2, ge, acc, esem, ssem):
    c, s, base = _worker_base()
    wid = s * NC + c

    pltpu.sync_copy(dst3_h.at[wid], idx2)

    _acc_init(acc, ge[0], K, s)
    plsc.subcore_barrier()

    def issue_in(sl, ch):
        b0 = base + ch * K
        pltpu.async_copy(e_h.at[pl.ds(b0, K)], ge[sl], esem[sl])

    def wait_in(sl, ch):
        b0 = base + ch * K
        pltpu.make_async_copy(e_h.at[pl.ds(b0, K)], ge[sl], esem[sl]).wait()

    def issue_scatter(sl, ch):
        pltpu.async_copy(ge[sl], acc.at[idx2.at[ch]], ssem[sl], add=True)

    def wait_scatter(sl, ch):
        pltpu.make_async_copy(ge[sl], acc.at[idx2.at[ch]], ssem[sl]).wait()

    for sl in (0, 1):
        issue_in(sl, sl)
    for sl in (0, 1):
        wait_in(sl, sl)
        issue_scatter(sl, sl)

    @pl.loop(2, NCHUNK, step=2)
    def _(i):
        for sl in (0, 1):
            wait_scatter(sl, i + sl - 2)
            issue_in(sl, i + sl)
        for sl in (0, 1):
            wait_in(sl, i + sl)
            issue_scatter(sl, i + sl)

    for sl in (0, 1):
        wait_scatter(sl, NCHUNK - 2 + sl)

    plsc.subcore_barrier()
    _acc_writeout(acc, msg_h, K, c, s)


KQ = 1000
NQCH = EPW2 // KQ


@functools.partial(
    pl.kernel,
    mesh=_mesh,
    out_type=jax.ShapeDtypeStruct((EH,), jnp.float32),
    compiler_params=pltpu.CompilerParams(needs_layout_passes=False),
    scratch_types=[
        pltpu.VMEM((N,), jnp.float32),
        pltpu.VMEM((N,), jnp.float32),
        pltpu.VMEM((16,), jnp.float32),
        pltpu.VMEM((KQ,), jnp.int32),
        pltpu.VMEM((KQ,), jnp.int32),
        pltpu.VMEM((KQ,), jnp.float32),
    ],
)
def _sc_passq(ps_h, pd_h, src_h, dst_h, out0_h, bpe_h,
              q_h,
              pst, pdt, bpev, idx_s, idx_d, qb):
    c = lax.axis_index("c")
    s = lax.axis_index("s")
    base = (s * NC + c) * EPW2

    pltpu.sync_copy(ps_h, pst)
    pltpu.sync_copy(pd_h, pdt)
    pltpu.sync_copy(bpe_h, bpev)

    @pl.loop(0, NQCH)
    def _(i):
        b0 = base + i * KQ
        pltpu.sync_copy(src_h.at[pl.ds(b0, KQ)], idx_s)
        pltpu.sync_copy(dst_h.at[pl.ds(b0, KQ)], idx_d)
        pltpu.sync_copy(out0_h.at[pl.ds(b0, KQ)], qb)

        @pl.loop(0, KQ // 16)
        def _(j):
            vs = idx_s[pl.ds(16 * j, 16)]
            vd = idx_d[pl.ds(16 * j, 16)]
            va = plsc.load_gather(pst, [vs])
            vb = plsc.load_gather(pdt, [vd])
            vo = qb[pl.ds(16 * j, 16)]
            qb[pl.ds(16 * j, 16)] = va + vb + vo + bpev[pl.ds(0, 16)]

        pltpu.sync_copy(qb, q_h.at[pl.ds(b0, KQ)])


def _tc_node_pre(nodes, W_ne, b_ne, W_ee, W_en):
    def body(nodes_ref, wne_ref, bne_ref, wee_ref, wen_ref,
             n0_ref, ns_ref, nd_ref, a_ref, b_ref):
        n0 = jnp.maximum(
            jnp.dot(nodes_ref[...], wne_ref[...],
                    preferred_element_type=jnp.float32) + bne_ref[...], 0.0)
        n0_ref[...] = n0
        ns_ref[...] = jnp.dot(n0, wee_ref[0:H, :],
                              preferred_element_type=jnp.float32)
        nd_ref[...] = jnp.dot(n0, wee_ref[H:2 * H, :],
                              preferred_element_type=jnp.float32)
        a_ref[...] = jnp.dot(n0, wen_ref[0:H, :],
                             preferred_element_type=jnp.float32)
        b_ref[...] = jnp.dot(n0, wen_ref[H:2 * H, :],
                             preferred_element_type=jnp.float32)

    shp = jax.ShapeDtypeStruct((N, H), jnp.float32)
    return pl.pallas_call(
        body,
        out_shape=[shp, shp, shp, shp, shp],
    )(nodes, W_ne, b_ne.reshape(1, H), W_ee, W_en)


def _tc_node_update(n, msgP, W_nn, b_nn, W_en):
    def body(n_ref, msg_ref, wnn_ref, bnn_ref, wen_ref,
             n1_ref, a_ref, b_ref):
        msg = msg_ref[0, 0:N, :] + msg_ref[1, 0:N, :]
        h = jnp.maximum(
            jnp.dot(n_ref[...], wnn_ref[0:H, :],
                    preferred_element_type=jnp.float32)
            + jnp.dot(msg, wnn_ref[H:2 * H, :],
                      preferred_element_type=jnp.float32)
            + bnn_ref[...], 0.0) + n_ref[...]
        n1_ref[...] = h
        a_ref[...] = jnp.dot(h, wen_ref[0:H, :],
                             preferred_element_type=jnp.float32)
        b_ref[...] = jnp.dot(h, wen_ref[H:2 * H, :],
                             preferred_element_type=jnp.float32)

    shp = jax.ShapeDtypeStruct((N, H), jnp.float32)
    return pl.pallas_call(
        body,
        out_shape=[shp, shp, shp],
    )(n, msgP, W_nn, b_nn.reshape(1, H), W_en)


def _tc_node_update2(n, msgP, W_nn, b_nn, Wpe_sd):
    def body(n_ref, msg_ref, wnn_ref, bnn_ref, wpe_ref, pspd_ref):
        msg = msg_ref[0, 0:N, :] + msg_ref[1, 0:N, :]
        h = jnp.maximum(
            jnp.dot(n_ref[...], wnn_ref[0:H, :],
                    preferred_element_type=jnp.float32)
            + jnp.dot(msg, wnn_ref[H:2 * H, :],
                      preferred_element_type=jnp.float32)
            + bnn_ref[...], 0.0) + n_ref[...]
        pspd_ref[...] = jnp.dot(h, wpe_ref[...],
                                preferred_element_type=jnp.float32)

    return pl.pallas_call(
        body,
        out_shape=jax.ShapeDtypeStruct((N, 8), jnp.float32),
    )(n, msgP, W_nn, b_nn.reshape(1, H), Wpe_sd)


BR = 2000


def _tc_edge_combine(e, z, W, b):
    def body(e_ref, z_ref, w_ref, b_ref, o_ref):
        o_ref[...] = jnp.maximum(
            z_ref[...] + jnp.dot(e_ref[...], w_ref[...],
                                 preferred_element_type=jnp.float32)
            + b_ref[...], 0.0) + e_ref[...]

    return pl.pallas_call(
        body,
        grid=(E // BR,),
        in_specs=[
            pl.BlockSpec((BR, H), lambda i: (i, 0)),
            pl.BlockSpec((BR, H), lambda i: (i, 0)),
            pl.BlockSpec((H, H), lambda i: (0, 0)),
            pl.BlockSpec((1, H), lambda i: (0, 0)),
        ],
        out_specs=pl.BlockSpec((BR, H), lambda i: (i, 0)),
        out_shape=jax.ShapeDtypeStruct((E, H), jnp.float32),
    )(e, z, W, b.reshape(1, H))


def _tc_final_half(e, z, W, b, wpe, half):
    boff = half * (EH // BR)

    def body(e_ref, z_ref, w_ref, b_ref, wpe_ref, o_ref):
        e3 = jnp.maximum(
            z_ref[...] + jnp.dot(e_ref[...], w_ref[...],
                                 preferred_element_type=jnp.float32)
            + b_ref[...], 0.0) + e_ref[...]
        o_ref[...] = jnp.sum(e3 * wpe_ref[...], axis=1, keepdims=True)

    return pl.pallas_call(
        body,
        grid=(EH // BR,),
        in_specs=[
            pl.BlockSpec((BR, H), lambda i: (i + boff, 0)),
            pl.BlockSpec((BR, H), lambda i: (i, 0)),
            pl.BlockSpec((H, H), lambda i: (0, 0)),
            pl.BlockSpec((1, H), lambda i: (0, 0)),
            pl.BlockSpec((1, H), lambda i: (0, 0)),
        ],
        out_specs=pl.BlockSpec((BR, 1), lambda i: (i, 0)),
        out_shape=jax.ShapeDtypeStruct((EH, 1), jnp.float32),
    )(e, z, W, b.reshape(1, H), wpe.reshape(1, H))


def kernel(nodes, start_index, end_index, W_ne, b_ne, W_ee, b_ee,
           W_nn, b_nn, W_en, b_en, W_pe, b_pe):
    src = start_index.astype(jnp.int32)
    dst = end_index.astype(jnp.int32)

    n0, ns, nd, A0, B0 = _tc_node_pre(nodes, W_ne, b_ne, W_ee, W_en)
    e1 = _sc_encode(ns, nd, src, dst, b_ee)
    z2 = _sc_gadd(A0, B0, src, dst)
    dst3 = dst.reshape(NW, NCHUNK, K)
    msg1 = _sc_scatter(e1, dst3)
    n1, A1, B1 = _tc_node_update(n0, msg1, W_nn, b_nn, W_en)
    e2 = _tc_edge_combine(e1, z2, W_en[2 * H:], b_en)
    src_a, src_b = src[0:EH], src[EH:E]
    dst_a, dst_b = dst[0:EH], dst[EH:E]
    z3a = _sc_gadd_half(A1, B1, src_a, dst_a)
    z3b = _sc_gadd_half(A1, B1, src_b, dst_b)
    msg2 = _sc_scatter(e2, dst3)
    out0a = _tc_final_half(e2, z3a, W_en[2 * H:], b_en, W_pe[2 * H:, 0], 0)
    out0b = _tc_final_half(e2, z3b, W_en[2 * H:], b_en, W_pe[2 * H:, 0], 1)
    Wpe_sd = jnp.concatenate(
        [W_pe[0:H], W_pe[H:2 * H], jnp.zeros((H, 6), jnp.float32)], axis=1)
    pspd = _tc_node_update2(n1, msg2, W_nn, b_nn, Wpe_sd)
    bpe16 = jnp.broadcast_to(b_pe, (16,))
    ps, pd = pspd[:, 0], pspd[:, 1]
    qa = _sc_passq(ps, pd, src_a, dst_a, out0a[:, 0], bpe16)
    qb = _sc_passq(ps, pd, src_b, dst_b, out0b[:, 0], bpe16)
    return jnp.concatenate([qa, qb])


# speedup vs baseline: 2.0445x; 1.0044x over previous
import functools

import jax
import jax.numpy as jnp
from jax import lax
from jax.experimental import pallas as pl
from jax.experimental.pallas import tpu as pltpu
from jax.experimental.pallas import tpu_sc as plsc

N = 10000
E = 320000
H = 128

NC = 2
NS = 16
NW = NC * NS
EPW = E // NW
K = 40
NCHUNK = EPW // K
SLOTS = 5
NP = 10240
RPS = NP // NS

_mesh = plsc.VectorSubcoreMesh(core_axis_name="c", subcore_axis_name="s")


def _worker_base():
    c = lax.axis_index("c")
    s = lax.axis_index("s")
    return c, s, (s * NC + c) * EPW


def _zero_rows(buf, rows):
    zv = jnp.zeros((16,), jnp.float32)

    @pl.loop(0, rows)
    def _(r):
        for c in range(H // 16):
            buf[r, pl.ds(16 * c, 16)] = zv


def _acc_init(acc, zbuf, rows, s):
    _zero_rows(zbuf, rows)
    base = s * RPS
    for j in range(RPS // rows):
        pltpu.sync_copy(zbuf, acc.at[pl.ds(base + j * rows, rows)])
    rem = RPS % rows
    if rem:
        pltpu.sync_copy(zbuf.at[pl.ds(0, rem)],
                        acc.at[pl.ds(base + (RPS // rows) * rows, rem)])


def _acc_writeout(acc, msg_hbm, rows, c, s):
    base = s * RPS
    for j in range(RPS // rows):
        pltpu.sync_copy(acc.at[pl.ds(base + j * rows, rows)],
                        msg_hbm.at[c, pl.ds(base + j * rows, rows)])
    rem = RPS % rows
    if rem:
        pltpu.sync_copy(acc.at[pl.ds(base + (RPS // rows) * rows, rem)],
                        msg_hbm.at[c, pl.ds(base + (RPS // rows) * rows, rem)])


def _make_gather_pass(with_bias, epw=EPW, nchunk=NCHUNK):
    scratch = [
        pltpu.VMEM((epw,), jnp.int32),
        pltpu.VMEM((epw,), jnp.int32),
        [pltpu.VMEM((K, H), jnp.float32)] * SLOTS,
        [pltpu.VMEM((K, H), jnp.float32)] * SLOTS,
        [pltpu.VMEM((K, H), jnp.float32)] * SLOTS,
        [pltpu.SemaphoreType.DMA] * SLOTS,
        [pltpu.SemaphoreType.DMA] * SLOTS,
    ]
    if with_bias:
        scratch.append(pltpu.VMEM((H,), jnp.float32))

    @functools.partial(
        pl.kernel,
        mesh=_mesh,
        out_type=jax.ShapeDtypeStruct((epw * NW, H), jnp.float32),
        scratch_types=scratch,
    )
    def gather_pass(a_h, b_h, src_h, dst_h, *rest):
        if with_bias:
            bee_h, o_h, idx_s, idx_d, ga, gb, oz, gsem, osem, bias_v = rest
        else:
            o_h, idx_s, idx_d, ga, gb, oz, gsem, osem = rest
        c = lax.axis_index("c")
        s = lax.axis_index("s")
        base = (s * NC + c) * epw

        if with_bias:
            pltpu.sync_copy(bee_h, bias_v)
            bvs = [bias_v[pl.ds(16 * cc, 16)] for cc in range(H // 16)]
        pltpu.sync_copy(src_h.at[pl.ds(base, epw)], idx_s)
        pltpu.sync_copy(dst_h.at[pl.ds(base, epw)], idx_d)

        def issue_in(sl, ch):
            o0 = ch * K
            pltpu.async_copy(a_h.at[idx_s.at[pl.ds(o0, K)]], ga[sl],
                             gsem[sl])
            pltpu.async_copy(b_h.at[idx_d.at[pl.ds(o0, K)]], gb[sl],
                             gsem[sl])

        def wait_in(sl, ch):
            o0 = ch * K
            pltpu.make_async_copy(a_h.at[idx_s.at[pl.ds(o0, K)]], ga[sl],
                                  gsem[sl]).wait()
            pltpu.make_async_copy(b_h.at[idx_d.at[pl.ds(o0, K)]], gb[sl],
                                  gsem[sl]).wait()

        def compute(sl):
            @pl.loop(0, K)
            def _(r):
                for cc in range(H // 16):
                    o = 16 * cc
                    va = ga[sl][r, pl.ds(o, 16)]
                    vb = gb[sl][r, pl.ds(o, 16)]
                    if with_bias:
                        oz[sl][r, pl.ds(o, 16)] = jnp.maximum(
                            va + vb + bvs[cc], 0.0)
                    else:
                        oz[sl][r, pl.ds(o, 16)] = va + vb

        def issue_out(sl, ch):
            b0 = base + ch * K
            pltpu.async_copy(oz[sl], o_h.at[pl.ds(b0, K)], osem[sl])

        def wait_out(sl, ch):
            b0 = base + ch * K
            pltpu.make_async_copy(oz[sl], o_h.at[pl.ds(b0, K)],
                                  osem[sl]).wait()

        slots = tuple(range(SLOTS))
        for sl in slots:
            issue_in(sl, sl)
        for sl in slots:
            wait_in(sl, sl)
            compute(sl)
            issue_out(sl, sl)
            issue_in(sl, sl + SLOTS)

        @pl.loop(SLOTS, nchunk - SLOTS, step=SLOTS)
        def _(i):
            for sl in slots:
                ch = i + sl
                wait_out(sl, ch - SLOTS)
                wait_in(sl, ch)
                compute(sl)
                issue_out(sl, ch)
                issue_in(sl, ch + SLOTS)

        for sl in slots:
            ch = nchunk - SLOTS + sl
            wait_out(sl, ch - SLOTS)
            wait_in(sl, ch)
            compute(sl)
            issue_out(sl, ch)
        for sl in slots:
            wait_out(sl, nchunk - SLOTS + sl)

    return gather_pass


EH = E // 2
EPW2 = EH // NW
NCHUNK2 = EPW2 // K

_sc_encode = _make_gather_pass(True)
_sc_gadd = _make_gather_pass(False)
_sc_gadd_half = _make_gather_pass(False, EPW2, NCHUNK2)


@functools.partial(
    pl.kernel,
    mesh=_mesh,
    out_type=jax.ShapeDtypeStruct((NC, NP, H), jnp.float32),
    scratch_types=[
        pltpu.VMEM((NCHUNK, K), jnp.int32),
        [pltpu.VMEM((K, H), jnp.float32)] * 2,
        pltpu.VMEM_SHARED((NP, H), jnp.float32),
        [pltpu.SemaphoreType.DMA] * 2,
        [pltpu.SemaphoreType.DMA] * 2,
    ],
)
def _sc_scatter(e_h, dst3_h,
                msg_h,
                idx2, ge, acc, esem, ssem):
    c, s, base = _worker_base()
    wid = s * NC + c

    pltpu.sync_copy(dst3_h.at[wid], idx2)

    _acc_init(acc, ge[0], K, s)
    plsc.subcore_barrier()

    def issue_in(sl, ch):
        b0 = base + ch * K
        pltpu.async_copy(e_h.at[pl.ds(b0, K)], ge[sl], esem[sl])

    def wait_in(sl, ch):
        b0 = base + ch * K
        pltpu.make_async_copy(e_h.at[pl.ds(b0, K)], ge[sl], esem[sl]).wait()

    def issue_scatter(sl, ch):
        pltpu.async_copy(ge[sl], acc.at[idx2.at[ch]], ssem[sl], add=True)

    def wait_scatter(sl, ch):
        pltpu.make_async_copy(ge[sl], acc.at[idx2.at[ch]], ssem[sl]).wait()

    for sl in (0, 1):
        issue_in(sl, sl)
    for sl in (0, 1):
        wait_in(sl, sl)
        issue_scatter(sl, sl)

    @pl.loop(2, NCHUNK, step=2)
    def _(i):
        for sl in (0, 1):
            wait_scatter(sl, i + sl - 2)
            issue_in(sl, i + sl)
        for sl in (0, 1):
            wait_in(sl, i + sl)
            issue_scatter(sl, i + sl)

    for sl in (0, 1):
        wait_scatter(sl, NCHUNK - 2 + sl)

    plsc.subcore_barrier()
    _acc_writeout(acc, msg_h, K, c, s)


NQB = EPW2 // 16


@functools.partial(
    pl.kernel,
    mesh=_mesh,
    out_type=jax.ShapeDtypeStruct((EH,), jnp.float32),
    compiler_params=pltpu.CompilerParams(needs_layout_passes=False),
    scratch_types=[
        pltpu.VMEM((N,), jnp.float32),
        pltpu.VMEM((N,), jnp.float32),
        pltpu.VMEM((16,), jnp.float32),
        pltpu.VMEM((EPW2,), jnp.int32),
        pltpu.VMEM((EPW2,), jnp.int32),
        pltpu.VMEM((EPW2,), jnp.float32),
        pltpu.VMEM((EPW2,), jnp.float32),
    ],
)
def _sc_passq(ps_h, pd_h, src_h, dst_h, out0_h, bpe_h,
              q_h,
              pst, pdt, bpev, idx_s, idx_d, ob, qb):
    c = lax.axis_index("c")
    s = lax.axis_index("s")
    base = (s * NC + c) * EPW2

    pltpu.sync_copy(ps_h, pst)
    pltpu.sync_copy(pd_h, pdt)
    pltpu.sync_copy(bpe_h, bpev)
    pltpu.sync_copy(src_h.at[pl.ds(base, EPW2)], idx_s)
    pltpu.sync_copy(dst_h.at[pl.ds(base, EPW2)], idx_d)
    pltpu.sync_copy(out0_h.at[pl.ds(base, EPW2)], ob)

    bv = bpev[pl.ds(0, 16)]

    def block(o):
        vs = idx_s[pl.ds(o, 16)]
        vd = idx_d[pl.ds(o, 16)]
        va = plsc.load_gather(pst, [vs])
        vb = plsc.load_gather(pdt, [vd])
        vo = ob[pl.ds(o, 16)]
        qb[pl.ds(o, 16)] = va + vb + vo + bv

    @pl.loop(0, NQB)
    def _(j):
        block(16 * j)

    block(EPW2 - 16)

    pltpu.sync_copy(qb, q_h.at[pl.ds(base, EPW2)])


def _tc_node_pre(nodes, W_ne, b_ne, W_ee, W_en):
    def body(nodes_ref, wne_ref, bne_ref, wee_ref, wen_ref,
             n0_ref, ns_ref, nd_ref, a_ref, b_ref):
        n0 = jnp.maximum(
            jnp.dot(nodes_ref[...], wne_ref[...],
                    preferred_element_type=jnp.float32) + bne_ref[...], 0.0)
        n0_ref[...] = n0
        ns_ref[...] = jnp.dot(n0, wee_ref[0:H, :],
                              preferred_element_type=jnp.float32)
        nd_ref[...] = jnp.dot(n0, wee_ref[H:2 * H, :],
                              preferred_element_type=jnp.float32)
        a_ref[...] = jnp.dot(n0, wen_ref[0:H, :],
                             preferred_element_type=jnp.float32)
        b_ref[...] = jnp.dot(n0, wen_ref[H:2 * H, :],
                             preferred_element_type=jnp.float32)

    shp = jax.ShapeDtypeStruct((N, H), jnp.float32)
    return pl.pallas_call(
        body,
        out_shape=[shp, shp, shp, shp, shp],
    )(nodes, W_ne, b_ne.reshape(1, H), W_ee, W_en)


def _tc_node_update(n, msgP, W_nn, b_nn, W_en):
    def body(n_ref, msg_ref, wnn_ref, bnn_ref, wen_ref,
             n1_ref, a_ref, b_ref):
        msg = msg_ref[0, 0:N, :] + msg_ref[1, 0:N, :]
        h = jnp.maximum(
            jnp.dot(n_ref[...], wnn_ref[0:H, :],
                    preferred_element_type=jnp.float32)
            + jnp.dot(msg, wnn_ref[H:2 * H, :],
                      preferred_element_type=jnp.float32)
            + bnn_ref[...], 0.0) + n_ref[...]
        n1_ref[...] = h
        a_ref[...] = jnp.dot(h, wen_ref[0:H, :],
                             preferred_element_type=jnp.float32)
        b_ref[...] = jnp.dot(h, wen_ref[H:2 * H, :],
                             preferred_element_type=jnp.float32)

    shp = jax.ShapeDtypeStruct((N, H), jnp.float32)
    return pl.pallas_call(
        body,
        out_shape=[shp, shp, shp],
    )(n, msgP, W_nn, b_nn.reshape(1, H), W_en)


def _tc_node_update2(n, msgP, W_nn, b_nn, Wpe_sd):
    def body(n_ref, msg_ref, wnn_ref, bnn_ref, wpe_ref, pspd_ref):
        msg = msg_ref[0, 0:N, :] + msg_ref[1, 0:N, :]
        h = jnp.maximum(
            jnp.dot(n_ref[...], wnn_ref[0:H, :],
                    preferred_element_type=jnp.float32)
            + jnp.dot(msg, wnn_ref[H:2 * H, :],
                      preferred_element_type=jnp.float32)
            + bnn_ref[...], 0.0) + n_ref[...]
        pspd_ref[...] = jnp.dot(h, wpe_ref[...],
                                preferred_element_type=jnp.float32)

    return pl.pallas_call(
        body,
        out_shape=jax.ShapeDtypeStruct((N, 8), jnp.float32),
    )(n, msgP, W_nn, b_nn.reshape(1, H), Wpe_sd)


BR = 2000


def _tc_edge_combine(e, z, W, b):
    def body(e_ref, z_ref, w_ref, b_ref, o_ref):
        o_ref[...] = jnp.maximum(
            z_ref[...] + jnp.dot(e_ref[...], w_ref[...],
                                 preferred_element_type=jnp.float32)
            + b_ref[...], 0.0) + e_ref[...]

    return pl.pallas_call(
        body,
        grid=(E // BR,),
        in_specs=[
            pl.BlockSpec((BR, H), lambda i: (i, 0)),
            pl.BlockSpec((BR, H), lambda i: (i, 0)),
            pl.BlockSpec((H, H), lambda i: (0, 0)),
            pl.BlockSpec((1, H), lambda i: (0, 0)),
        ],
        out_specs=pl.BlockSpec((BR, H), lambda i: (i, 0)),
        out_shape=jax.ShapeDtypeStruct((E, H), jnp.float32),
    )(e, z, W, b.reshape(1, H))


def _tc_final_half(e, z, W, b, wpe, half):
    boff = half * (EH // BR)

    def body(e_ref, z_ref, w_ref, b_ref, wpe_ref, o_ref):
        e3 = jnp.maximum(
            z_ref[...] + jnp.dot(e_ref[...], w_ref[...],
                                 preferred_element_type=jnp.float32)
            + b_ref[...], 0.0) + e_ref[...]
        o_ref[...] = jnp.sum(e3 * wpe_ref[...], axis=1, keepdims=True)

    return pl.pallas_call(
        body,
        grid=(EH // BR,),
        in_specs=[
            pl.BlockSpec((BR, H), lambda i: (i + boff, 0)),
            pl.BlockSpec((BR, H), lambda i: (i, 0)),
            pl.BlockSpec((H, H), lambda i: (0, 0)),
            pl.BlockSpec((1, H), lambda i: (0, 0)),
            pl.BlockSpec((1, H), lambda i: (0, 0)),
        ],
        out_specs=pl.BlockSpec((BR, 1), lambda i: (i, 0)),
        out_shape=jax.ShapeDtypeStruct((EH, 1), jnp.float32),
    )(e, z, W, b.reshape(1, H), wpe.reshape(1, H))


def kernel(nodes, start_index, end_index, W_ne, b_ne, W_ee, b_ee,
           W_nn, b_nn, W_en, b_en, W_pe, b_pe):
    src = start_index.astype(jnp.int32)
    dst = end_index.astype(jnp.int32)

    n0, ns, nd, A0, B0 = _tc_node_pre(nodes, W_ne, b_ne, W_ee, W_en)
    e1 = _sc_encode(ns, nd, src, dst, b_ee)
    z2 = _sc_gadd(A0, B0, src, dst)
    dst3 = dst.reshape(NW, NCHUNK, K)
    msg1 = _sc_scatter(e1, dst3)
    n1, A1, B1 = _tc_node_update(n0, msg1, W_nn, b_nn, W_en)
    e2 = _tc_edge_combine(e1, z2, W_en[2 * H:], b_en)
    src_a, src_b = src[0:EH], src[EH:E]
    dst_a, dst_b = dst[0:EH], dst[EH:E]
    z3a = _sc_gadd_half(A1, B1, src_a, dst_a)
    z3b = _sc_gadd_half(A1, B1, src_b, dst_b)
    msg2 = _sc_scatter(e2, dst3)
    out0a = _tc_final_half(e2, z3a, W_en[2 * H:], b_en, W_pe[2 * H:, 0], 0)
    out0b = _tc_final_half(e2, z3b, W_en[2 * H:], b_en, W_pe[2 * H:, 0], 1)
    Wpe_sd = jnp.concatenate(
        [W_pe[0:H], W_pe[H:2 * H], jnp.zeros((H, 6), jnp.float32)], axis=1)
    pspd = _tc_node_update2(n1, msg2, W_nn, b_nn, Wpe_sd)
    bpe16 = jnp.broadcast_to(b_pe, (16,))
    ps, pd = pspd[:, 0], pspd[:, 1]
    qa = _sc_passq(ps, pd, src_a, dst_a, out0a[:, 0], bpe16)
    qb = _sc_passq(ps, pd, src_b, dst_b, out0b[:, 0], bpe16)
    return jnp.concatenate([qa, qb])
